# Initial kernel scaffold; baseline (speedup 1.0000x reference)
import functools

import jax
import jax.numpy as jnp
from jax import lax
from jax.experimental import pallas as pl
from jax.experimental.pallas import tpu as pltpu
from jax.experimental.pallas import tpu_sc as plsc

N = 10000
E = 320000
D0, D1, D2 = 128, 128, 64

NC = 2
NS = 16
NW = NC * NS
EPW = E // NW
CH = 80
NCH = EPW // CH
NPAD = 10240
RPT = NPAD // NS
KB = RPT // CH

_mesh = plsc.VectorSubcoreMesh(core_axis_name="c", subcore_axis_name="s")


@functools.partial(
    pl.kernel,
    out_type=jax.ShapeDtypeStruct((NC * NPAD,), jnp.float32),
    mesh=_mesh,
    scratch_types=[
        pltpu.VMEM((NCH, CH), jnp.int32),
        pltpu.VMEM((CH,), jnp.float32),
        pltpu.VMEM((RPT,), jnp.float32),
        pltpu.VMEM_SHARED((NPAD,), jnp.float32),
    ],
)
def _hist(dst_hbm, out_hbm, dst_v, ones_v, stage_v, acc):
    cid = lax.axis_index("c")
    sid = lax.axis_index("s")
    wid = sid * NC + cid
    for i in range(CH // 16):
        ones_v[pl.ds(i * 16, 16)] = jnp.ones((16,), jnp.float32)
    for i in range(RPT // 16):
        stage_v[pl.ds(i * 16, 16)] = jnp.zeros((16,), jnp.float32)
    pltpu.sync_copy(stage_v, acc.at[pl.ds(sid * RPT, RPT)])
    pltpu.sync_copy(dst_hbm.at[wid], dst_v)
    plsc.subcore_barrier()

    def body(j, carry):
        pltpu.sync_copy(ones_v, acc.at[dst_v.at[j]], add=True)
        return carry

    lax.fori_loop(0, NCH, body, 0)
    plsc.subcore_barrier()
    pltpu.sync_copy(acc.at[pl.ds(sid * RPT, RPT)], stage_v)
    pltpu.sync_copy(stage_v, out_hbm.at[pl.ds(cid * NPAD + sid * RPT, RPT)])


def _make_agg(F):
    @functools.partial(
        pl.kernel,
        out_type=jax.ShapeDtypeStruct((NC * NPAD, F), jnp.float32),
        mesh=_mesh,
        scratch_types=[
            pltpu.VMEM((NCH, CH), jnp.int32),
            pltpu.VMEM((NCH, CH), jnp.int32),
            pltpu.VMEM((CH, F), jnp.float32),
            pltpu.VMEM((CH, F), jnp.float32),
            pltpu.VMEM_SHARED((NPAD, F), jnp.float32),
            pltpu.SemaphoreType.DMA,
        ],
    )
    def agg(h_hbm, src_hbm, dst_hbm, out_hbm, src_v, dst_v, rows_v, stage_v, acc, sem):
        cid = lax.axis_index("c")
        sid = lax.axis_index("s")
        wid = sid * NC + cid
        pltpu.sync_copy(src_hbm.at[wid], src_v)
        pltpu.sync_copy(dst_hbm.at[wid], dst_v)

        def zrow(r, carry):
            for c in range(F // 16):
                stage_v[r, pl.ds(c * 16, 16)] = jnp.zeros((16,), jnp.float32)
            return carry

        lax.fori_loop(0, CH, zrow, 0)
        for k in range(KB):
            pltpu.sync_copy(stage_v, acc.at[pl.ds(sid * RPT + k * CH, CH)])
        plsc.subcore_barrier()

        def body(j, carry):
            pltpu.async_copy(h_hbm.at[src_v.at[j]], rows_v, sem).wait()
            pltpu.sync_copy(rows_v, acc.at[dst_v.at[j]], add=True)
            return carry

        lax.fori_loop(0, NCH, body, 0)
        plsc.subcore_barrier()
        for k in range(KB):
            off = sid * RPT + k * CH
            pltpu.sync_copy(acc.at[pl.ds(off, CH)], stage_v)
            pltpu.sync_copy(stage_v, out_hbm.at[pl.ds(cid * NPAD + off, CH)])

    return agg


_agg128 = _make_agg(D1)
_agg64 = _make_agg(D2)

_B = 1000
_G = N // _B


def _rowblock(i):
    return (i, 0)


def _tc1_body(c0, c1, x, w, o):
    dinv = lax.rsqrt(c0[...] + c1[...] + 1.0)
    o[...] = jnp.dot(x[...], w[...], preferred_element_type=jnp.float32) * dinv


_tc1 = pl.pallas_call(
    _tc1_body,
    grid=(_G,),
    in_specs=[
        pl.BlockSpec((_B, 1), _rowblock),
        pl.BlockSpec((_B, 1), _rowblock),
        pl.BlockSpec((_B, D0), _rowblock),
        pl.BlockSpec((D0, D1), lambda i: (0, 0)),
    ],
    out_specs=pl.BlockSpec((_B, D1), _rowblock),
    out_shape=jax.ShapeDtypeStruct((N, D1), jnp.float32),
)


def _tc2_body(c0, c1, p0, p1, hp, b, w, o):
    dinv = lax.rsqrt(c0[...] + c1[...] + 1.0)
    y = jnp.maximum(dinv * (p0[...] + p1[...] + hp[...]) + b[...], 0.0)
    o[...] = jnp.dot(y, w[...], preferred_element_type=jnp.float32) * dinv


_tc2 = pl.pallas_call(
    _tc2_body,
    grid=(_G,),
    in_specs=[
        pl.BlockSpec((_B, 1), _rowblock),
        pl.BlockSpec((_B, 1), _rowblock),
        pl.BlockSpec((_B, D1), _rowblock),
        pl.BlockSpec((_B, D1), _rowblock),
        pl.BlockSpec((_B, D1), _rowblock),
        pl.BlockSpec((1, D1), lambda i: (0, 0)),
        pl.BlockSpec((D1, D2), lambda i: (0, 0)),
    ],
    out_specs=pl.BlockSpec((_B, D2), _rowblock),
    out_shape=jax.ShapeDtypeStruct((N, D2), jnp.float32),
)


def _tc3_body(c0, c1, q0, q1, hp, b, o):
    dinv = lax.rsqrt(c0[...] + c1[...] + 1.0)
    o[...] = dinv * (q0[...] + q1[...] + hp[...]) + b[...]


_tc3 = pl.pallas_call(
    _tc3_body,
    grid=(_G,),
    in_specs=[
        pl.BlockSpec((_B, 1), _rowblock),
        pl.BlockSpec((_B, 1), _rowblock),
        pl.BlockSpec((_B, D2), _rowblock),
        pl.BlockSpec((_B, D2), _rowblock),
        pl.BlockSpec((_B, D2), _rowblock),
        pl.BlockSpec((1, D2), lambda i: (0, 0)),
    ],
    out_specs=pl.BlockSpec((_B, D2), _rowblock),
    out_shape=jax.ShapeDtypeStruct((N, D2), jnp.float32),
)


def kernel(x, edge_index, W1, b1, W2, b2):
    src = edge_index[0].reshape(NW, NCH, CH)
    dst = edge_index[1].reshape(NW, NCH, CH)

    hist = _hist(dst)
    c0 = hist[:N].reshape(N, 1)
    c1 = hist[NPAD:NPAD + N].reshape(N, 1)

    h1p = _tc1(c0, c1, x, W1)
    p = _agg128(h1p, src, dst)
    h2p = _tc2(c0, c1, p[:N], p[NPAD:NPAD + N], h1p, b1.reshape(1, D1), W2)
    q = _agg64(h2p, src, dst)
    out = _tc3(c0, c1, q[:N], q[NPAD:NPAD + N], h2p, b2.reshape(1, D2))
    return out


# speedup vs baseline: 20.8338x; 20.8338x over previous
import functools

import jax
import jax.numpy as jnp
from jax import lax
from jax.experimental import pallas as pl
from jax.experimental.pallas import tpu as pltpu
from jax.experimental.pallas import tpu_sc as plsc

N = 10000
E = 320000
D0, D1, D2 = 128, 128, 64

NC = 2
NS = 16
NW = NC * NS
EPW = E // NW
CH = 80
NCH = EPW // CH
NPAD = 10240
RPT = NPAD // NS
KB = RPT // CH

_mesh = plsc.VectorSubcoreMesh(core_axis_name="c", subcore_axis_name="s")
_sc_params = pltpu.CompilerParams(use_tc_tiling_on_sc=False)


@functools.partial(
    pl.kernel,
    out_type=jax.ShapeDtypeStruct((NC * NPAD,), jnp.float32),
    mesh=_mesh,
    scratch_types=[
        pltpu.VMEM((NCH, CH), jnp.int32),
        pltpu.VMEM((CH,), jnp.float32),
        pltpu.VMEM((RPT,), jnp.float32),
        pltpu.VMEM_SHARED((NPAD,), jnp.float32),
    ],
    compiler_params=_sc_params,
)
def _hist(dst_hbm, out_hbm, dst_v, ones_v, stage_v, acc):
    cid = lax.axis_index("c")
    sid = lax.axis_index("s")
    wid = sid * NC + cid
    for i in range(CH // 16):
        ones_v[pl.ds(i * 16, 16)] = jnp.ones((16,), jnp.float32)
    for i in range(RPT // 16):
        stage_v[pl.ds(i * 16, 16)] = jnp.zeros((16,), jnp.float32)
    pltpu.sync_copy(stage_v, acc.at[pl.ds(sid * RPT, RPT)])
    pltpu.sync_copy(dst_hbm.at[wid], dst_v)
    plsc.subcore_barrier()

    def body(j, carry):
        pltpu.sync_copy(ones_v, acc.at[dst_v.at[j]], add=True)
        return carry

    lax.fori_loop(0, NCH, body, 0)
    plsc.subcore_barrier()
    pltpu.sync_copy(acc.at[pl.ds(sid * RPT, RPT)], stage_v)
    pltpu.sync_copy(stage_v, out_hbm.at[pl.ds(cid * NPAD + sid * RPT, RPT)])


def _make_agg(F):
    @functools.partial(
        pl.kernel,
        out_type=jax.ShapeDtypeStruct((NC * NPAD, F), jnp.float32),
        mesh=_mesh,
        scratch_types=[
            pltpu.VMEM((NCH, CH), jnp.int32),
            pltpu.VMEM((NCH, CH), jnp.int32),
            pltpu.VMEM((CH, F), jnp.float32),
            pltpu.VMEM((CH, F), jnp.float32),
            pltpu.VMEM_SHARED((NPAD, F), jnp.float32),
            pltpu.SemaphoreType.DMA,
        ],
        compiler_params=_sc_params,
    )
    def agg(h_hbm, src_hbm, dst_hbm, out_hbm, src_v, dst_v, rows_v, stage_v, acc, sem):
        cid = lax.axis_index("c")
        sid = lax.axis_index("s")
        wid = sid * NC + cid
        pltpu.sync_copy(src_hbm.at[wid], src_v)
        pltpu.sync_copy(dst_hbm.at[wid], dst_v)

        def zrow(r, carry):
            for c in range(F // 16):
                stage_v[r, pl.ds(c * 16, 16)] = jnp.zeros((16,), jnp.float32)
            return carry

        lax.fori_loop(0, CH, zrow, 0)
        for k in range(KB):
            pltpu.sync_copy(stage_v, acc.at[pl.ds(sid * RPT + k * CH, CH)])
        plsc.subcore_barrier()

        def body(j, carry):
            pltpu.async_copy(h_hbm.at[src_v.at[j]], rows_v, sem).wait()
            pltpu.sync_copy(rows_v, acc.at[dst_v.at[j]], add=True)
            return carry

        lax.fori_loop(0, NCH, body, 0)
        plsc.subcore_barrier()
        for k in range(KB):
            off = sid * RPT + k * CH
            pltpu.sync_copy(acc.at[pl.ds(off, CH)], stage_v)
            pltpu.sync_copy(stage_v, out_hbm.at[pl.ds(cid * NPAD + off, CH)])

    return agg


_agg128 = _make_agg(D1)
_agg64 = _make_agg(D2)

_B = 1000
_G = N // _B


def _rowblock(i):
    return (i, 0)


def _tc1_body(c0, c1, x, w, o):
    dinv = lax.rsqrt(c0[...] + c1[...] + 1.0)
    o[...] = jnp.dot(x[...], w[...], preferred_element_type=jnp.float32) * dinv


_tc1 = pl.pallas_call(
    _tc1_body,
    grid=(_G,),
    in_specs=[
        pl.BlockSpec((_B, 1), _rowblock),
        pl.BlockSpec((_B, 1), _rowblock),
        pl.BlockSpec((_B, D0), _rowblock),
        pl.BlockSpec((D0, D1), lambda i: (0, 0)),
    ],
    out_specs=pl.BlockSpec((_B, D1), _rowblock),
    out_shape=jax.ShapeDtypeStruct((N, D1), jnp.float32),
)


def _tc2_body(c0, c1, p0, p1, hp, b, w, o):
    dinv = lax.rsqrt(c0[...] + c1[...] + 1.0)
    y = jnp.maximum(dinv * (p0[...] + p1[...] + hp[...]) + b[...], 0.0)
    o[...] = jnp.dot(y, w[...], preferred_element_type=jnp.float32) * dinv


_tc2 = pl.pallas_call(
    _tc2_body,
    grid=(_G,),
    in_specs=[
        pl.BlockSpec((_B, 1), _rowblock),
        pl.BlockSpec((_B, 1), _rowblock),
        pl.BlockSpec((_B, D1), _rowblock),
        pl.BlockSpec((_B, D1), _rowblock),
        pl.BlockSpec((_B, D1), _rowblock),
        pl.BlockSpec((1, D1), lambda i: (0, 0)),
        pl.BlockSpec((D1, D2), lambda i: (0, 0)),
    ],
    out_specs=pl.BlockSpec((_B, D2), _rowblock),
    out_shape=jax.ShapeDtypeStruct((N, D2), jnp.float32),
)


def _tc3_body(c0, c1, q0, q1, hp, b, o):
    dinv = lax.rsqrt(c0[...] + c1[...] + 1.0)
    o[...] = dinv * (q0[...] + q1[...] + hp[...]) + b[...]


_tc3 = pl.pallas_call(
    _tc3_body,
    grid=(_G,),
    in_specs=[
        pl.BlockSpec((_B, 1), _rowblock),
        pl.BlockSpec((_B, 1), _rowblock),
        pl.BlockSpec((_B, D2), _rowblock),
        pl.BlockSpec((_B, D2), _rowblock),
        pl.BlockSpec((_B, D2), _rowblock),
        pl.BlockSpec((1, D2), lambda i: (0, 0)),
    ],
    out_specs=pl.BlockSpec((_B, D2), _rowblock),
    out_shape=jax.ShapeDtypeStruct((N, D2), jnp.float32),
)


def kernel(x, edge_index, W1, b1, W2, b2):
    src = edge_index[0].reshape(NW, NCH, CH)
    dst = edge_index[1].reshape(NW, NCH, CH)

    hist = _hist(dst)
    c0 = hist[:N].reshape(N, 1)
    c1 = hist[NPAD:NPAD + N].reshape(N, 1)

    h1p = _tc1(c0, c1, x, W1)
    p = _agg128(h1p, src, dst)
    h2p = _tc2(c0, c1, p[:N], p[NPAD:NPAD + N], h1p, b1.reshape(1, D1), W2)
    q = _agg64(h2p, src, dst)
    out = _tc3(c0, c1, q[:N], q[NPAD:NPAD + N], h2p, b2.reshape(1, D2))
    return out


# speedup vs baseline: 32.8729x; 1.5779x over previous
import functools

import jax
import jax.numpy as jnp
from jax import lax
from jax.experimental import pallas as pl
from jax.experimental.pallas import tpu as pltpu
from jax.experimental.pallas import tpu_sc as plsc

N = 10000
E = 320000
D0, D1, D2 = 128, 128, 64
FH = D1 // 2

NC = 2
NS = 16
NW = NC * NS
CH = 128
NCH = 80
NCH2 = 160
EPAD = NW * NCH * CH
NPAD = 10240
RPT = NPAD // NS
KB = RPT // CH
K = 4

_mesh = plsc.VectorSubcoreMesh(core_axis_name="c", subcore_axis_name="s")
_sc_params = pltpu.CompilerParams(use_tc_tiling_on_sc=False)


@functools.partial(
    pl.kernel,
    out_type=jax.ShapeDtypeStruct((NC * NPAD,), jnp.float32),
    mesh=_mesh,
    scratch_types=[
        pltpu.VMEM((NCH, CH), jnp.int32),
        pltpu.VMEM((CH,), jnp.float32),
        pltpu.VMEM((RPT,), jnp.float32),
        pltpu.VMEM_SHARED((NPAD,), jnp.float32),
    ],
    compiler_params=_sc_params,
)
def _hist(dst_hbm, out_hbm, dst_v, ones_v, stage_v, acc):
    cid = lax.axis_index("c")
    sid = lax.axis_index("s")
    wid = sid * NC + cid
    for i in range(CH // 16):
        ones_v[pl.ds(i * 16, 16)] = jnp.ones((16,), jnp.float32)
    for i in range(RPT // 16):
        stage_v[pl.ds(i * 16, 16)] = jnp.zeros((16,), jnp.float32)
    pltpu.sync_copy(stage_v, acc.at[pl.ds(sid * RPT, RPT)])
    pltpu.sync_copy(dst_hbm.at[wid], dst_v)
    plsc.subcore_barrier()

    def body(j, carry):
        pltpu.sync_copy(ones_v, acc.at[dst_v.at[j]], add=True)
        return carry

    lax.fori_loop(0, NCH, body, 0)
    plsc.subcore_barrier()
    pltpu.sync_copy(acc.at[pl.ds(sid * RPT, RPT)], stage_v)
    pltpu.sync_copy(stage_v, out_hbm.at[pl.ds(cid * NPAD + sid * RPT, RPT)])


def _ring_loop(h_hbm, acc, src_v, dst_v, bufs, gsems, ssems, nch):
    for s in range(K):
        pltpu.async_copy(h_hbm.at[src_v.at[s]], bufs[s], gsems[s])

    def round_body(r, carry):
        j0 = r * K
        for s in range(K):
            j = j0 + s
            pltpu.make_async_copy(h_hbm.at[src_v.at[j]], bufs[s], gsems[s]).wait()
            pltpu.async_copy(bufs[s], acc.at[dst_v.at[j]], ssems[s], add=True)
        for s in range(K):
            jn = j0 + K + s

            @pl.when(jn < nch)
            def _():
                pltpu.make_async_copy(
                    bufs[s], acc.at[dst_v.at[j0 + s]], ssems[s]).wait()
                pltpu.async_copy(h_hbm.at[src_v.at[jn]], bufs[s], gsems[s])
        return carry

    lax.fori_loop(0, nch // K, round_body, 0)
    for s in range(K):
        pltpu.make_async_copy(
            bufs[s], acc.at[dst_v.at[nch - K + s]], ssems[s]).wait()


def _zero_acc(stage_v, acc, sid, fh):
    def zrow(r, carry):
        for c in range(fh // 16):
            stage_v[r, pl.ds(c * 16, 16)] = jnp.zeros((16,), jnp.float32)
        return carry

    lax.fori_loop(0, CH, zrow, 0)
    for k in range(KB):
        pltpu.sync_copy(stage_v, acc.at[pl.ds(sid * RPT + k * CH, CH)])


def _writeback(stage_v, acc, out_hbm, cid, sid):
    for k in range(KB):
        off = sid * RPT + k * CH
        pltpu.sync_copy(acc.at[pl.ds(off, CH)], stage_v)
        pltpu.sync_copy(stage_v, out_hbm.at[pl.ds(cid * NPAD + off, CH)])


@functools.partial(
    pl.kernel,
    out_type=jax.ShapeDtypeStruct((NC * NPAD, FH), jnp.float32),
    mesh=_mesh,
    scratch_types=[
        pltpu.VMEM((NCH2, CH), jnp.int32),
        pltpu.VMEM((NCH2, CH), jnp.int32),
        pltpu.VMEM((CH, FH), jnp.float32),
        pltpu.VMEM((CH, FH), jnp.float32),
        pltpu.VMEM((CH, FH), jnp.float32),
        pltpu.VMEM((CH, FH), jnp.float32),
        pltpu.VMEM_SHARED((NPAD, FH), jnp.float32),
        pltpu.SemaphoreType.DMA,
        pltpu.SemaphoreType.DMA,
        pltpu.SemaphoreType.DMA,
        pltpu.SemaphoreType.DMA,
        pltpu.SemaphoreType.DMA,
        pltpu.SemaphoreType.DMA,
        pltpu.SemaphoreType.DMA,
        pltpu.SemaphoreType.DMA,
    ],
    compiler_params=_sc_params,
)
def _agg1(h2n_hbm, srca_hbm, srcb_hbm, dst_hbm, out_hbm,
          src_v, dst_v, b0, b1, b2, b3, acc,
          g0, g1, g2, g3, s0, s1, s2, s3):
    bufs = (b0, b1, b2, b3)
    gsems = (g0, g1, g2, g3)
    ssems = (s0, s1, s2, s3)
    cid = lax.axis_index("c")
    sid = lax.axis_index("s")

    @pl.when(cid == 0)
    def _():
        pltpu.sync_copy(srca_hbm.at[sid], src_v)

    @pl.when(cid == 1)
    def _():
        pltpu.sync_copy(srcb_hbm.at[sid], src_v)

    pltpu.sync_copy(dst_hbm.at[sid], dst_v)
    _zero_acc(b0, acc, sid, FH)
    plsc.subcore_barrier()
    _ring_loop(h2n_hbm, acc, src_v, dst_v, bufs, gsems, ssems, NCH2)
    plsc.subcore_barrier()
    _writeback(b0, acc, out_hbm, cid, sid)


@functools.partial(
    pl.kernel,
    out_type=jax.ShapeDtypeStruct((NC * NPAD, D2), jnp.float32),
    mesh=_mesh,
    scratch_types=[
        pltpu.VMEM((NCH, CH), jnp.int32),
        pltpu.VMEM((NCH, CH), jnp.int32),
        pltpu.VMEM((CH, D2), jnp.float32),
        pltpu.VMEM((CH, D2), jnp.float32),
        pltpu.VMEM((CH, D2), jnp.float32),
        pltpu.VMEM((CH, D2), jnp.float32),
        pltpu.VMEM_SHARED((NPAD, D2), jnp.float32),
        pltpu.SemaphoreType.DMA,
        pltpu.SemaphoreType.DMA,
        pltpu.SemaphoreType.DMA,
        pltpu.SemaphoreType.DMA,
        pltpu.SemaphoreType.DMA,
        pltpu.SemaphoreType.DMA,
        pltpu.SemaphoreType.DMA,
        pltpu.SemaphoreType.DMA,
    ],
    compiler_params=_sc_params,
)
def _agg2(h_hbm, src_hbm, dst_hbm, out_hbm,
          src_v, dst_v, b0, b1, b2, b3, acc,
          g0, g1, g2, g3, s0, s1, s2, s3):
    bufs = (b0, b1, b2, b3)
    gsems = (g0, g1, g2, g3)
    ssems = (s0, s1, s2, s3)
    cid = lax.axis_index("c")
    sid = lax.axis_index("s")
    wid = sid * NC + cid
    pltpu.sync_copy(src_hbm.at[wid], src_v)
    pltpu.sync_copy(dst_hbm.at[wid], dst_v)
    _zero_acc(b0, acc, sid, D2)
    plsc.subcore_barrier()
    _ring_loop(h_hbm, acc, src_v, dst_v, bufs, gsems, ssems, NCH)
    plsc.subcore_barrier()
    _writeback(b0, acc, out_hbm, cid, sid)


_B = 1024
_G = pl.cdiv(N, _B)


def _rowblock(i):
    return (i, 0)


def _cblock(i):
    return (0, i, 0)


def _tc1_body(cnt, x, w, o):
    c = cnt[0] + cnt[1]
    dinv = lax.rsqrt(c + 1.0)
    h = jnp.dot(x[...], w[...], preferred_element_type=jnp.float32) * dinv
    o[0] = h[:, :FH]
    o[1] = h[:, FH:]


_tc1 = pl.pallas_call(
    _tc1_body,
    grid=(_G,),
    in_specs=[
        pl.BlockSpec((NC, _B, 1), _cblock),
        pl.BlockSpec((_B, D0), _rowblock),
        pl.BlockSpec((D0, D1), lambda i: (0, 0)),
    ],
    out_specs=pl.BlockSpec((NC, _B, FH), _cblock),
    out_shape=jax.ShapeDtypeStruct((NC, N, FH), jnp.float32),
)


def _tc2_body(cnt, p, hp, b, w, o):
    c = cnt[0] + cnt[1]
    dinv = lax.rsqrt(c + 1.0)
    za = dinv * (p[0] + hp[0])
    zb = dinv * (p[1] + hp[1])
    y = jnp.maximum(jnp.concatenate([za, zb], axis=1) + b[...], 0.0)
    o[...] = jnp.dot(y, w[...], preferred_element_type=jnp.float32) * dinv


_tc2 = pl.pallas_call(
    _tc2_body,
    grid=(_G,),
    in_specs=[
        pl.BlockSpec((NC, _B, 1), _cblock),
        pl.BlockSpec((NC, _B, FH), _cblock),
        pl.BlockSpec((NC, _B, FH), _cblock),
        pl.BlockSpec((1, D1), lambda i: (0, 0)),
        pl.BlockSpec((D1, D2), lambda i: (0, 0)),
    ],
    out_specs=pl.BlockSpec((_B, D2), _rowblock),
    out_shape=jax.ShapeDtypeStruct((N, D2), jnp.float32),
)


def _tc3_body(cnt, q, hp, b, o):
    c = cnt[0] + cnt[1]
    dinv = lax.rsqrt(c + 1.0)
    o[...] = dinv * (q[0] + q[1] + hp[...]) + b[...]


_tc3 = pl.pallas_call(
    _tc3_body,
    grid=(_G,),
    in_specs=[
        pl.BlockSpec((NC, _B, 1), _cblock),
        pl.BlockSpec((NC, _B, D2), _cblock),
        pl.BlockSpec((_B, D2), _rowblock),
        pl.BlockSpec((1, D2), lambda i: (0, 0)),
    ],
    out_specs=pl.BlockSpec((_B, D2), _rowblock),
    out_shape=jax.ShapeDtypeStruct((N, D2), jnp.float32),
)


def kernel(x, edge_index, W1, b1, W2, b2):
    npd = EPAD - E
    ar = jnp.arange(npd, dtype=jnp.int32)
    pad_src = ar % N
    pad_dst = N + ar % (NPAD - N)
    src = jnp.concatenate([edge_index[0], pad_src])
    dst = jnp.concatenate([edge_index[1], pad_dst])
    src32 = src.reshape(NW, NCH, CH)
    dst32 = dst.reshape(NW, NCH, CH)
    srca16 = src.reshape(NS, NCH2, CH)
    srcb16 = srca16 + N
    dst16 = dst.reshape(NS, NCH2, CH)

    cnt = _hist(dst32).reshape(NC, NPAD, 1)
    h1p = _tc1(cnt, x, W1)
    h2n = h1p.reshape(NC * N, FH)
    p = _agg1(h2n, srca16, srcb16, dst16).reshape(NC, NPAD, FH)
    h2p = _tc2(cnt, p, h1p, b1.reshape(1, D1), W2)
    q = _agg2(h2p, src32, dst32).reshape(NC, NPAD, D2)
    out = _tc3(cnt, q, h2p, b2.reshape(1, D2))
    return out


# speedup vs baseline: 36.7561x; 1.1181x over previous
import functools

import jax
import jax.numpy as jnp
from jax import lax
from jax.experimental import pallas as pl
from jax.experimental.pallas import tpu as pltpu
from jax.experimental.pallas import tpu_sc as plsc

N = 10000
E = 320000
D0, D1, D2 = 128, 128, 64
FH = D1 // 2

NC = 2
NS = 16
NW = NC * NS
CH = 128
NCH = 80
NCH2 = 160
EPAD = NW * NCH * CH
NPAD = 10240
RPT = NPAD // NS
KB = RPT // CH
K = 4

_mesh = plsc.VectorSubcoreMesh(core_axis_name="c", subcore_axis_name="s")
_sc_params = pltpu.CompilerParams(use_tc_tiling_on_sc=False)


@functools.partial(
    pl.kernel,
    out_type=jax.ShapeDtypeStruct((NC * NPAD,), jnp.float32),
    mesh=_mesh,
    scratch_types=[
        pltpu.VMEM((NCH, CH), jnp.int32),
        pltpu.VMEM((CH,), jnp.float32),
        pltpu.VMEM((RPT,), jnp.float32),
        pltpu.VMEM_SHARED((NPAD,), jnp.float32),
    ],
    compiler_params=_sc_params,
)
def _hist(dst_hbm, out_hbm, dst_v, ones_v, stage_v, acc):
    cid = lax.axis_index("c")
    sid = lax.axis_index("s")
    wid = sid * NC + cid
    for i in range(CH // 16):
        ones_v[pl.ds(i * 16, 16)] = jnp.ones((16,), jnp.float32)
    for i in range(RPT // 16):
        stage_v[pl.ds(i * 16, 16)] = jnp.zeros((16,), jnp.float32)
    pltpu.sync_copy(stage_v, acc.at[pl.ds(sid * RPT, RPT)])
    pltpu.sync_copy(dst_hbm.at[wid], dst_v)
    plsc.subcore_barrier()

    def body(j, carry):
        pltpu.sync_copy(ones_v, acc.at[dst_v.at[j]], add=True)
        return carry

    lax.fori_loop(0, NCH, body, 0)
    plsc.subcore_barrier()
    pltpu.sync_copy(acc.at[pl.ds(sid * RPT, RPT)], stage_v)
    pltpu.sync_copy(stage_v, out_hbm.at[pl.ds(cid * NPAD + sid * RPT, RPT)])


def _ring_loop(h_hbm, acc, src_v, dst_v, bufs, gsems, ssems, nch):
    for s in range(K):
        pltpu.async_copy(h_hbm.at[src_v.at[s]], bufs[s], gsems[s])

    def round_body(r, carry):
        j0 = r * K
        for s in range(K):
            j = j0 + s
            pltpu.make_async_copy(h_hbm.at[src_v.at[j]], bufs[s], gsems[s]).wait()
            pltpu.async_copy(bufs[s], acc.at[dst_v.at[j]], ssems[s], add=True)
        for s in range(K):
            jn = j0 + K + s

            @pl.when(jn < nch)
            def _():
                pltpu.make_async_copy(
                    bufs[s], acc.at[dst_v.at[j0 + s]], ssems[s]).wait()
                pltpu.async_copy(h_hbm.at[src_v.at[jn]], bufs[s], gsems[s])
        return carry

    lax.fori_loop(0, nch // K, round_body, 0)
    for s in range(K):
        pltpu.make_async_copy(
            bufs[s], acc.at[dst_v.at[nch - K + s]], ssems[s]).wait()


def _zero_acc(stage_v, acc, sid, fh):
    def zrow(r, carry):
        for c in range(fh // 16):
            stage_v[r, pl.ds(c * 16, 16)] = jnp.zeros((16,), jnp.float32)
        return carry

    lax.fori_loop(0, CH, zrow, 0)
    for k in range(KB):
        pltpu.sync_copy(stage_v, acc.at[pl.ds(sid * RPT + k * CH, CH)])


def _fill_wb_idx(idx_wb, cid, sid):
    for k in range(KB):
        for i in range(CH // 16):
            r = sid * RPT + k * CH + i * 16 + lax.iota(jnp.int32, 16)
            idx_wb[k, pl.ds(i * 16, 16)] = r * 2 + cid


def _writeback(stage_v, acc, out_hbm, idx_wb, sid):
    for k in range(KB):
        pltpu.sync_copy(acc.at[pl.ds(sid * RPT + k * CH, CH)], stage_v)
        pltpu.sync_copy(stage_v, out_hbm.at[idx_wb.at[k]])


@functools.partial(
    pl.kernel,
    out_type=jax.ShapeDtypeStruct((2 * NPAD, FH), jnp.float32),
    mesh=_mesh,
    scratch_types=[
        pltpu.VMEM((NCH2, CH), jnp.int32),
        pltpu.VMEM((NCH2, CH), jnp.int32),
        pltpu.VMEM((CH, FH), jnp.float32),
        pltpu.VMEM((CH, FH), jnp.float32),
        pltpu.VMEM((CH, FH), jnp.float32),
        pltpu.VMEM((CH, FH), jnp.float32),
        pltpu.VMEM((KB, CH), jnp.int32),
        pltpu.VMEM_SHARED((NPAD, FH), jnp.float32),
        pltpu.SemaphoreType.DMA,
        pltpu.SemaphoreType.DMA,
        pltpu.SemaphoreType.DMA,
        pltpu.SemaphoreType.DMA,
        pltpu.SemaphoreType.DMA,
        pltpu.SemaphoreType.DMA,
        pltpu.SemaphoreType.DMA,
        pltpu.SemaphoreType.DMA,
    ],
    compiler_params=_sc_params,
)
def _agg1(h2n_hbm, src_hbm, dst_hbm, out_hbm,
          src_v, dst_v, b0, b1, b2, b3, idx_wb, acc,
          g0, g1, g2, g3, s0, s1, s2, s3):
    bufs = (b0, b1, b2, b3)
    gsems = (g0, g1, g2, g3)
    ssems = (s0, s1, s2, s3)
    cid = lax.axis_index("c")
    sid = lax.axis_index("s")
    pltpu.sync_copy(src_hbm.at[sid], src_v)
    pltpu.sync_copy(dst_hbm.at[sid], dst_v)
    _fill_wb_idx(idx_wb, cid, sid)

    def xform(r, carry):
        for c in range(CH // 16):
            v = src_v[r, pl.ds(c * 16, 16)]
            src_v[r, pl.ds(c * 16, 16)] = v * 2 + cid
        return carry

    lax.fori_loop(0, NCH2, xform, 0)
    _zero_acc(b0, acc, sid, FH)
    plsc.subcore_barrier()
    _ring_loop(h2n_hbm, acc, src_v, dst_v, bufs, gsems, ssems, NCH2)
    plsc.subcore_barrier()
    _writeback(b0, acc, out_hbm, idx_wb, sid)


@functools.partial(
    pl.kernel,
    out_type=jax.ShapeDtypeStruct((2 * NPAD, D2), jnp.float32),
    mesh=_mesh,
    scratch_types=[
        pltpu.VMEM((NCH, CH), jnp.int32),
        pltpu.VMEM((NCH, CH), jnp.int32),
        pltpu.VMEM((CH, D2), jnp.float32),
        pltpu.VMEM((CH, D2), jnp.float32),
        pltpu.VMEM((CH, D2), jnp.float32),
        pltpu.VMEM((CH, D2), jnp.float32),
        pltpu.VMEM((KB, CH), jnp.int32),
        pltpu.VMEM_SHARED((NPAD, D2), jnp.float32),
        pltpu.SemaphoreType.DMA,
        pltpu.SemaphoreType.DMA,
        pltpu.SemaphoreType.DMA,
        pltpu.SemaphoreType.DMA,
        pltpu.SemaphoreType.DMA,
        pltpu.SemaphoreType.DMA,
        pltpu.SemaphoreType.DMA,
        pltpu.SemaphoreType.DMA,
    ],
    compiler_params=_sc_params,
)
def _agg2(h_hbm, src_hbm, dst_hbm, out_hbm,
          src_v, dst_v, b0, b1, b2, b3, idx_wb, acc,
          g0, g1, g2, g3, s0, s1, s2, s3):
    bufs = (b0, b1, b2, b3)
    gsems = (g0, g1, g2, g3)
    ssems = (s0, s1, s2, s3)
    cid = lax.axis_index("c")
    sid = lax.axis_index("s")
    wid = sid * NC + cid
    pltpu.sync_copy(src_hbm.at[wid], src_v)
    pltpu.sync_copy(dst_hbm.at[wid], dst_v)
    _fill_wb_idx(idx_wb, cid, sid)
    _zero_acc(b0, acc, sid, D2)
    plsc.subcore_barrier()
    _ring_loop(h_hbm, acc, src_v, dst_v, bufs, gsems, ssems, NCH)
    plsc.subcore_barrier()
    _writeback(b0, acc, out_hbm, idx_wb, sid)


_B = 1024
_G = pl.cdiv(N, _B)


def _rowblock(i):
    return (i, 0)


def _cblock(i):
    return (0, i, 0)


def _tc1_body(cnt, x, w, o):
    c = cnt[0] + cnt[1]
    dinv = lax.rsqrt(c + 1.0)
    o[...] = jnp.dot(x[...], w[...], preferred_element_type=jnp.float32) * dinv


_tc1 = pl.pallas_call(
    _tc1_body,
    grid=(_G,),
    in_specs=[
        pl.BlockSpec((NC, _B, 1), _cblock),
        pl.BlockSpec((_B, D0), _rowblock),
        pl.BlockSpec((D0, D1), lambda i: (0, 0)),
    ],
    out_specs=pl.BlockSpec((_B, D1), _rowblock),
    out_shape=jax.ShapeDtypeStruct((N, D1), jnp.float32),
)


def _tc2_body(cnt, p, hp, b, w, o):
    c = cnt[0] + cnt[1]
    dinv = lax.rsqrt(c + 1.0)
    y = jnp.maximum(dinv * (p[...] + hp[...]) + b[...], 0.0)
    o[...] = jnp.dot(y, w[...], preferred_element_type=jnp.float32) * dinv


_tc2 = pl.pallas_call(
    _tc2_body,
    grid=(_G,),
    in_specs=[
        pl.BlockSpec((NC, _B, 1), _cblock),
        pl.BlockSpec((_B, D1), _rowblock),
        pl.BlockSpec((_B, D1), _rowblock),
        pl.BlockSpec((1, D1), lambda i: (0, 0)),
        pl.BlockSpec((D1, D2), lambda i: (0, 0)),
    ],
    out_specs=pl.BlockSpec((_B, D2), _rowblock),
    out_shape=jax.ShapeDtypeStruct((N, D2), jnp.float32),
)


def _tc3_body(cnt, q, hp, b, o):
    c = cnt[0] + cnt[1]
    dinv = lax.rsqrt(c + 1.0)
    o[...] = dinv * (q[:, :D2] + q[:, D2:] + hp[...]) + b[...]


_tc3 = pl.pallas_call(
    _tc3_body,
    grid=(_G,),
    in_specs=[
        pl.BlockSpec((NC, _B, 1), _cblock),
        pl.BlockSpec((_B, 2 * D2), _rowblock),
        pl.BlockSpec((_B, D2), _rowblock),
        pl.BlockSpec((1, D2), lambda i: (0, 0)),
    ],
    out_specs=pl.BlockSpec((_B, D2), _rowblock),
    out_shape=jax.ShapeDtypeStruct((N, D2), jnp.float32),
)


def kernel(x, edge_index, W1, b1, W2, b2):
    npd = EPAD - E
    ar = jnp.arange(npd, dtype=jnp.int32)
    pad_src = ar % N
    pad_dst = N + ar % (NPAD - N)
    src = jnp.concatenate([edge_index[0], pad_src])
    dst = jnp.concatenate([edge_index[1], pad_dst])
    src32 = src.reshape(NW, NCH, CH)
    dst32 = dst.reshape(NW, NCH, CH)
    src16 = src.reshape(NS, NCH2, CH)
    dst16 = dst.reshape(NS, NCH2, CH)

    cnt = _hist(dst32).reshape(NC, NPAD, 1)
    h1p = _tc1(cnt, x, W1)
    h2n = h1p.reshape(2 * N, FH)
    p = _agg1(h2n, src16, dst16).reshape(NPAD, D1)
    h2p = _tc2(cnt, p, h1p, b1.reshape(1, D1), W2)
    q = _agg2(h2p, src32, dst32).reshape(NPAD, 2 * D2)
    out = _tc3(cnt, q, h2p, b2.reshape(1, D2))
    return out


# speedup vs baseline: 38.3381x; 1.0430x over previous
import functools

import jax
import jax.numpy as jnp
from jax import lax
from jax.experimental import pallas as pl
from jax.experimental.pallas import tpu as pltpu
from jax.experimental.pallas import tpu_sc as plsc

N = 10000
E = 320000
D0, D1, D2 = 128, 128, 64
FH = D1 // 2

NC = 2
NS = 16
NW = NC * NS
CH = 128
NCH = 80
NCH2 = 160
EPAD = NW * NCH * CH
NPAD = 10240
RPT = NPAD // NS
KB = RPT // CH
K = 4

_mesh = plsc.VectorSubcoreMesh(core_axis_name="c", subcore_axis_name="s")
_sc_params = pltpu.CompilerParams(use_tc_tiling_on_sc=False)


NRC = NPAD // NC
RPT2 = NRC // NS
GB = 80


@functools.partial(
    pl.kernel,
    out_type=jax.ShapeDtypeStruct((NPAD, D1), jnp.float32),
    mesh=_mesh,
    scratch_types=[
        pltpu.VMEM((NCH2, CH), jnp.int32),
        pltpu.VMEM((CH,), jnp.float32),
        pltpu.VMEM((RPT,), jnp.float32),
        pltpu.VMEM((RPT2,), jnp.float32),
        pltpu.VMEM((GB, D1), jnp.float32),
        pltpu.VMEM_SHARED((NPAD,), jnp.float32),
        pltpu.SemaphoreType.DMA,
    ],
    compiler_params=_sc_params,
)
def _hist(dst_hbm, out_hbm, dst_v, ones_v, zero_v, cnt_v, stage_v, acc, sem):
    cid = lax.axis_index("c")
    sid = lax.axis_index("s")
    for i in range(CH // 16):
        ones_v[pl.ds(i * 16, 16)] = jnp.ones((16,), jnp.float32)
    for i in range(RPT // 16):
        zero_v[pl.ds(i * 16, 16)] = jnp.zeros((16,), jnp.float32)
    pltpu.sync_copy(zero_v, acc.at[pl.ds(sid * RPT, RPT)])
    pltpu.sync_copy(dst_hbm.at[sid], dst_v)
    plsc.subcore_barrier()

    def body(j, carry):
        pltpu.async_copy(ones_v, acc.at[dst_v.at[j]], sem, add=True)
        return carry

    lax.fori_loop(0, NCH2, body, 0)

    def drain(j, carry):
        pltpu.make_async_copy(ones_v, acc.at[dst_v.at[0]], sem).wait()
        return carry

    lax.fori_loop(0, NCH2, drain, 0)
    plsc.subcore_barrier()
    row0 = cid * NRC + sid * RPT2
    pltpu.sync_copy(acc.at[pl.ds(row0, RPT2)], cnt_v)
    for g in range(RPT2 // GB):
        def brow(r16, carry):
            c16 = cnt_v[pl.ds(g * GB + r16 * 16, 16)]
            for l in range(16):
                row = r16 * 16 + l
                v = jnp.full((16,), c16[l], jnp.float32)
                for i in range(D1 // 16):
                    stage_v[row, pl.ds(i * 16, 16)] = v
            return carry

        lax.fori_loop(0, GB // 16, brow, 0)
        pltpu.sync_copy(stage_v, out_hbm.at[pl.ds(row0 + g * GB, GB)])


def _ring_loop(h_hbm, acc, src_v, dst_v, bufs, gsems, ssems, nch):
    for s in range(K):
        pltpu.async_copy(h_hbm.at[src_v.at[s]], bufs[s], gsems[s])

    def round_body(r, carry):
        j0 = r * K
        for s in range(K):
            j = j0 + s
            pltpu.make_async_copy(h_hbm.at[src_v.at[j]], bufs[s], gsems[s]).wait()
            pltpu.async_copy(bufs[s], acc.at[dst_v.at[j]], ssems[s], add=True)
        for s in range(K):
            jn = j0 + K + s

            @pl.when(jn < nch)
            def _():
                pltpu.make_async_copy(
                    bufs[s], acc.at[dst_v.at[j0 + s]], ssems[s]).wait()
                pltpu.async_copy(h_hbm.at[src_v.at[jn]], bufs[s], gsems[s])
        return carry

    lax.fori_loop(0, nch // K, round_body, 0)
    for s in range(K):
        pltpu.make_async_copy(
            bufs[s], acc.at[dst_v.at[nch - K + s]], ssems[s]).wait()


def _zero_acc(stage_v, acc, sid, fh):
    def zrow(r, carry):
        for c in range(fh // 16):
            stage_v[r, pl.ds(c * 16, 16)] = jnp.zeros((16,), jnp.float32)
        return carry

    lax.fori_loop(0, CH, zrow, 0)
    for k in range(KB):
        pltpu.sync_copy(stage_v, acc.at[pl.ds(sid * RPT + k * CH, CH)])


def _fill_wb_idx(idx_wb, cid, sid):
    for k in range(KB):
        for i in range(CH // 16):
            r = sid * RPT + k * CH + i * 16 + lax.iota(jnp.int32, 16)
            idx_wb[k, pl.ds(i * 16, 16)] = r * 2 + cid


def _writeback(stage_v, acc, out_hbm, idx_wb, sid):
    for k in range(KB):
        pltpu.sync_copy(acc.at[pl.ds(sid * RPT + k * CH, CH)], stage_v)
        pltpu.sync_copy(stage_v, out_hbm.at[idx_wb.at[k]])


@functools.partial(
    pl.kernel,
    out_type=jax.ShapeDtypeStruct((2 * NPAD, FH), jnp.float32),
    mesh=_mesh,
    scratch_types=[
        pltpu.VMEM((NCH2, CH), jnp.int32),
        pltpu.VMEM((NCH2, CH), jnp.int32),
        pltpu.VMEM((CH, FH), jnp.float32),
        pltpu.VMEM((CH, FH), jnp.float32),
        pltpu.VMEM((CH, FH), jnp.float32),
        pltpu.VMEM((CH, FH), jnp.float32),
        pltpu.VMEM((KB, CH), jnp.int32),
        pltpu.VMEM_SHARED((NPAD, FH), jnp.float32),
        pltpu.SemaphoreType.DMA,
        pltpu.SemaphoreType.DMA,
        pltpu.SemaphoreType.DMA,
        pltpu.SemaphoreType.DMA,
        pltpu.SemaphoreType.DMA,
        pltpu.SemaphoreType.DMA,
        pltpu.SemaphoreType.DMA,
        pltpu.SemaphoreType.DMA,
    ],
    compiler_params=_sc_params,
)
def _agg1(h2n_hbm, src_hbm, dst_hbm, out_hbm,
          src_v, dst_v, b0, b1, b2, b3, idx_wb, acc,
          g0, g1, g2, g3, s0, s1, s2, s3):
    bufs = (b0, b1, b2, b3)
    gsems = (g0, g1, g2, g3)
    ssems = (s0, s1, s2, s3)
    cid = lax.axis_index("c")
    sid = lax.axis_index("s")
    pltpu.sync_copy(src_hbm.at[sid], src_v)
    pltpu.sync_copy(dst_hbm.at[sid], dst_v)
    _fill_wb_idx(idx_wb, cid, sid)

    def xform(r, carry):
        for c in range(CH // 16):
            v = src_v[r, pl.ds(c * 16, 16)]
            src_v[r, pl.ds(c * 16, 16)] = v * 2 + cid
        return carry

    lax.fori_loop(0, NCH2, xform, 0)
    _zero_acc(b0, acc, sid, FH)
    plsc.subcore_barrier()
    _ring_loop(h2n_hbm, acc, src_v, dst_v, bufs, gsems, ssems, NCH2)
    plsc.subcore_barrier()
    _writeback(b0, acc, out_hbm, idx_wb, sid)


@functools.partial(
    pl.kernel,
    out_type=jax.ShapeDtypeStruct((2 * NPAD, D2), jnp.float32),
    mesh=_mesh,
    scratch_types=[
        pltpu.VMEM((NCH, CH), jnp.int32),
        pltpu.VMEM((NCH, CH), jnp.int32),
        pltpu.VMEM((CH, D2), jnp.float32),
        pltpu.VMEM((CH, D2), jnp.float32),
        pltpu.VMEM((CH, D2), jnp.float32),
        pltpu.VMEM((CH, D2), jnp.float32),
        pltpu.VMEM((KB, CH), jnp.int32),
        pltpu.VMEM_SHARED((NPAD, D2), jnp.float32),
        pltpu.SemaphoreType.DMA,
        pltpu.SemaphoreType.DMA,
        pltpu.SemaphoreType.DMA,
        pltpu.SemaphoreType.DMA,
        pltpu.SemaphoreType.DMA,
        pltpu.SemaphoreType.DMA,
        pltpu.SemaphoreType.DMA,
        pltpu.SemaphoreType.DMA,
    ],
    compiler_params=_sc_params,
)
def _agg2(h_hbm, src_hbm, dst_hbm, out_hbm,
          src_v, dst_v, b0, b1, b2, b3, idx_wb, acc,
          g0, g1, g2, g3, s0, s1, s2, s3):
    bufs = (b0, b1, b2, b3)
    gsems = (g0, g1, g2, g3)
    ssems = (s0, s1, s2, s3)
    cid = lax.axis_index("c")
    sid = lax.axis_index("s")
    wid = sid * NC + cid
    pltpu.sync_copy(src_hbm.at[wid], src_v)
    pltpu.sync_copy(dst_hbm.at[wid], dst_v)
    _fill_wb_idx(idx_wb, cid, sid)
    _zero_acc(b0, acc, sid, D2)
    plsc.subcore_barrier()
    _ring_loop(h_hbm, acc, src_v, dst_v, bufs, gsems, ssems, NCH)
    plsc.subcore_barrier()
    _writeback(b0, acc, out_hbm, idx_wb, sid)


_B = 1024
_G = pl.cdiv(N, _B)


def _rowblock(i):
    return (i, 0)


def _cblock(i):
    return (0, i, 0)


def _tc1_body(cnt, x, w, o):
    dinv = lax.rsqrt(cnt[...] + 1.0)
    o[...] = jnp.dot(x[...], w[...], preferred_element_type=jnp.float32) * dinv


_tc1 = pl.pallas_call(
    _tc1_body,
    grid=(_G,),
    in_specs=[
        pl.BlockSpec((_B, D1), _rowblock),
        pl.BlockSpec((_B, D0), _rowblock),
        pl.BlockSpec((D0, D1), lambda i: (0, 0)),
    ],
    out_specs=pl.BlockSpec((_B, D1), _rowblock),
    out_shape=jax.ShapeDtypeStruct((N, D1), jnp.float32),
)


def _tc2_body(cnt, p, hp, b, w, o):
    dinv = lax.rsqrt(cnt[...] + 1.0)
    y = jnp.maximum(dinv * (p[...] + hp[...]) + b[...], 0.0)
    o[...] = jnp.dot(y, w[...], preferred_element_type=jnp.float32) * dinv[:, :D2]


_tc2 = pl.pallas_call(
    _tc2_body,
    grid=(_G,),
    in_specs=[
        pl.BlockSpec((_B, D1), _rowblock),
        pl.BlockSpec((_B, D1), _rowblock),
        pl.BlockSpec((_B, D1), _rowblock),
        pl.BlockSpec((1, D1), lambda i: (0, 0)),
        pl.BlockSpec((D1, D2), lambda i: (0, 0)),
    ],
    out_specs=pl.BlockSpec((_B, D2), _rowblock),
    out_shape=jax.ShapeDtypeStruct((N, D2), jnp.float32),
)


def _tc3_body(cnt, q, hp, b, o):
    dinv = lax.rsqrt(cnt[...] + 1.0)
    o[...] = dinv[:, :D2] * (q[:, :D2] + q[:, D2:] + hp[...]) + b[...]


_tc3 = pl.pallas_call(
    _tc3_body,
    grid=(_G,),
    in_specs=[
        pl.BlockSpec((_B, D1), _rowblock),
        pl.BlockSpec((_B, 2 * D2), _rowblock),
        pl.BlockSpec((_B, D2), _rowblock),
        pl.BlockSpec((1, D2), lambda i: (0, 0)),
    ],
    out_specs=pl.BlockSpec((_B, D2), _rowblock),
    out_shape=jax.ShapeDtypeStruct((N, D2), jnp.float32),
)


def kernel(x, edge_index, W1, b1, W2, b2):
    npd = EPAD - E
    ar = jnp.arange(npd, dtype=jnp.int32)
    pad_src = ar % N
    pad_dst = N + ar % (NPAD - N)
    src = jnp.concatenate([edge_index[0], pad_src])
    dst = jnp.concatenate([edge_index[1], pad_dst])
    src32 = src.reshape(NW, NCH, CH)
    dst32 = dst.reshape(NW, NCH, CH)
    src16 = src.reshape(NS, NCH2, CH)
    dst16 = dst.reshape(NS, NCH2, CH)

    cnt = _hist(dst16)
    h1p = _tc1(cnt, x, W1)
    h2n = h1p.reshape(2 * N, FH)
    p = _agg1(h2n, src16, dst16).reshape(NPAD, D1)
    h2p = _tc2(cnt, p, h1p, b1.reshape(1, D1), W2)
    q = _agg2(h2p, src32, dst32).reshape(NPAD, 2 * D2)
    out = _tc3(cnt, q, h2p, b2.reshape(1, D2))
    return out


# speedup vs baseline: 39.2558x; 1.0239x over previous
import functools

import jax
import jax.numpy as jnp
from jax import lax
from jax.experimental import pallas as pl
from jax.experimental.pallas import tpu as pltpu
from jax.experimental.pallas import tpu_sc as plsc

N = 10000
E = 320000
D0, D1, D2 = 128, 128, 64
FH = D1 // 2

NC = 2
NS = 16
NW = NC * NS
CH = 128
NCH = 80
NCH2 = 160
EC = NW * NCH
ECR = E // CH
NPAD = 10240
RPT = NPAD // NS
KB = RPT // CH
K = 4

_mesh = plsc.VectorSubcoreMesh(core_axis_name="c", subcore_axis_name="s")
_sc_params = pltpu.CompilerParams(use_tc_tiling_on_sc=False)

_PEB = 256


def _prep_body(ei, o):
    o[...] = ei[...].reshape(2, _PEB, CH)


_prep = pl.pallas_call(
    _prep_body,
    grid=(EC // _PEB,),
    in_specs=[pl.BlockSpec((2, _PEB * CH), lambda c: (0, c))],
    out_specs=pl.BlockSpec((2, _PEB, CH), lambda c: (0, c, 0)),
    out_shape=jax.ShapeDtypeStruct((2, EC, CH), jnp.int32),
)


def _fix_pad_rows(idx_v, lo, hi, base_row, is_dst):
    for r in range(lo, hi):
        for i in range(CH // 16):
            k = (base_row + r) * CH + i * 16 + lax.iota(jnp.int32, 16)
            if is_dst:
                idx_v[r, pl.ds(i * 16, 16)] = N + k % (NPAD - N)
            else:
                idx_v[r, pl.ds(i * 16, 16)] = k % N


NRC = NPAD // NC
RPT2 = NRC // NS
GB = 80


@functools.partial(
    pl.kernel,
    out_type=jax.ShapeDtypeStruct((NPAD, D1), jnp.float32),
    mesh=_mesh,
    scratch_types=[
        pltpu.VMEM((NCH2, CH), jnp.int32),
        pltpu.VMEM((CH,), jnp.float32),
        pltpu.VMEM((RPT,), jnp.float32),
        pltpu.VMEM((RPT2,), jnp.float32),
        pltpu.VMEM((GB, D1), jnp.float32),
        pltpu.VMEM_SHARED((NPAD,), jnp.float32),
        pltpu.SemaphoreType.DMA,
    ],
    compiler_params=_sc_params,
)
def _hist(ei_hbm, out_hbm, dst_v, ones_v, zero_v, cnt_v, stage_v, acc, sem):
    cid = lax.axis_index("c")
    sid = lax.axis_index("s")
    for i in range(CH // 16):
        ones_v[pl.ds(i * 16, 16)] = jnp.ones((16,), jnp.float32)
    for i in range(RPT // 16):
        zero_v[pl.ds(i * 16, 16)] = jnp.zeros((16,), jnp.float32)
    pltpu.sync_copy(zero_v, acc.at[pl.ds(sid * RPT, RPT)])
    pltpu.sync_copy(ei_hbm.at[1, pl.ds(sid * NCH2, NCH2)], dst_v)

    @pl.when(sid == NS - 1)
    def _():
        _fix_pad_rows(dst_v, NCH2 - (EC - ECR), NCH2, (NS - 1) * NCH2, True)

    plsc.subcore_barrier()

    def body(j, carry):
        pltpu.async_copy(ones_v, acc.at[dst_v.at[j]], sem, add=True)
        return carry

    lax.fori_loop(0, NCH2, body, 0)

    def drain(j, carry):
        pltpu.make_async_copy(ones_v, acc.at[dst_v.at[0]], sem).wait()
        return carry

    lax.fori_loop(0, NCH2, drain, 0)
    plsc.subcore_barrier()
    row0 = cid * NRC + sid * RPT2
    pltpu.sync_copy(acc.at[pl.ds(row0, RPT2)], cnt_v)
    for g in range(RPT2 // GB):
        def brow(r16, carry):
            c16 = cnt_v[pl.ds(g * GB + r16 * 16, 16)]
            for l in range(16):
                row = r16 * 16 + l
                v = jnp.full((16,), c16[l], jnp.float32)
                for i in range(D1 // 16):
                    stage_v[row, pl.ds(i * 16, 16)] = v
            return carry

        lax.fori_loop(0, GB // 16, brow, 0)
        pltpu.sync_copy(stage_v, out_hbm.at[pl.ds(row0 + g * GB, GB)])


def _ring_loop(h_hbm, acc, src_v, dst_v, bufs, gsems, ssems, nch):
    for s in range(K):
        pltpu.async_copy(h_hbm.at[src_v.at[s]], bufs[s], gsems[s])

    def round_body(r, carry):
        j0 = r * K
        for s in range(K):
            j = j0 + s
            pltpu.make_async_copy(h_hbm.at[src_v.at[j]], bufs[s], gsems[s]).wait()
            pltpu.async_copy(bufs[s], acc.at[dst_v.at[j]], ssems[s], add=True)
        for s in range(K):
            jn = j0 + K + s

            @pl.when(jn < nch)
            def _():
                pltpu.make_async_copy(
                    bufs[s], acc.at[dst_v.at[j0 + s]], ssems[s]).wait()
                pltpu.async_copy(h_hbm.at[src_v.at[jn]], bufs[s], gsems[s])
        return carry

    lax.fori_loop(0, nch // K, round_body, 0)
    for s in range(K):
        pltpu.make_async_copy(
            bufs[s], acc.at[dst_v.at[nch - K + s]], ssems[s]).wait()


def _zero_acc(stage_v, acc, sid, fh):
    def zrow(r, carry):
        for c in range(fh // 16):
            stage_v[r, pl.ds(c * 16, 16)] = jnp.zeros((16,), jnp.float32)
        return carry

    lax.fori_loop(0, CH, zrow, 0)
    for k in range(KB):
        pltpu.sync_copy(stage_v, acc.at[pl.ds(sid * RPT + k * CH, CH)])


def _fill_wb_idx(idx_wb, cid, sid):
    for k in range(KB):
        for i in range(CH // 16):
            r = sid * RPT + k * CH + i * 16 + lax.iota(jnp.int32, 16)
            idx_wb[k, pl.ds(i * 16, 16)] = r * 2 + cid


def _writeback(stage_v, acc, out_hbm, idx_wb, sid):
    for k in range(KB):
        pltpu.sync_copy(acc.at[pl.ds(sid * RPT + k * CH, CH)], stage_v)
        pltpu.sync_copy(stage_v, out_hbm.at[idx_wb.at[k]])


@functools.partial(
    pl.kernel,
    out_type=jax.ShapeDtypeStruct((2 * NPAD, FH), jnp.float32),
    mesh=_mesh,
    scratch_types=[
        pltpu.VMEM((NCH2, CH), jnp.int32),
        pltpu.VMEM((NCH2, CH), jnp.int32),
        pltpu.VMEM((CH, FH), jnp.float32),
        pltpu.VMEM((CH, FH), jnp.float32),
        pltpu.VMEM((CH, FH), jnp.float32),
        pltpu.VMEM((CH, FH), jnp.float32),
        pltpu.VMEM((KB, CH), jnp.int32),
        pltpu.VMEM_SHARED((NPAD, FH), jnp.float32),
        pltpu.SemaphoreType.DMA,
        pltpu.SemaphoreType.DMA,
        pltpu.SemaphoreType.DMA,
        pltpu.SemaphoreType.DMA,
        pltpu.SemaphoreType.DMA,
        pltpu.SemaphoreType.DMA,
        pltpu.SemaphoreType.DMA,
        pltpu.SemaphoreType.DMA,
    ],
    compiler_params=_sc_params,
)
def _agg1(h2n_hbm, ei_hbm, out_hbm,
          src_v, dst_v, b0, b1, b2, b3, idx_wb, acc,
          g0, g1, g2, g3, s0, s1, s2, s3):
    bufs = (b0, b1, b2, b3)
    gsems = (g0, g1, g2, g3)
    ssems = (s0, s1, s2, s3)
    cid = lax.axis_index("c")
    sid = lax.axis_index("s")
    pltpu.sync_copy(ei_hbm.at[0, pl.ds(sid * NCH2, NCH2)], src_v)
    pltpu.sync_copy(ei_hbm.at[1, pl.ds(sid * NCH2, NCH2)], dst_v)

    @pl.when(sid == NS - 1)
    def _():
        _fix_pad_rows(src_v, NCH2 - (EC - ECR), NCH2, (NS - 1) * NCH2, False)
        _fix_pad_rows(dst_v, NCH2 - (EC - ECR), NCH2, (NS - 1) * NCH2, True)

    _fill_wb_idx(idx_wb, cid, sid)

    def xform(r, carry):
        for c in range(CH // 16):
            v = src_v[r, pl.ds(c * 16, 16)]
            src_v[r, pl.ds(c * 16, 16)] = v * 2 + cid
        return carry

    lax.fori_loop(0, NCH2, xform, 0)
    _zero_acc(b0, acc, sid, FH)
    plsc.subcore_barrier()
    _ring_loop(h2n_hbm, acc, src_v, dst_v, bufs, gsems, ssems, NCH2)
    plsc.subcore_barrier()
    _writeback(b0, acc, out_hbm, idx_wb, sid)


@functools.partial(
    pl.kernel,
    out_type=jax.ShapeDtypeStruct((2 * NPAD, D2), jnp.float32),
    mesh=_mesh,
    scratch_types=[
        pltpu.VMEM((NCH, CH), jnp.int32),
        pltpu.VMEM((NCH, CH), jnp.int32),
        pltpu.VMEM((CH, D2), jnp.float32),
        pltpu.VMEM((CH, D2), jnp.float32),
        pltpu.VMEM((CH, D2), jnp.float32),
        pltpu.VMEM((CH, D2), jnp.float32),
        pltpu.VMEM((KB, CH), jnp.int32),
        pltpu.VMEM_SHARED((NPAD, D2), jnp.float32),
        pltpu.SemaphoreType.DMA,
        pltpu.SemaphoreType.DMA,
        pltpu.SemaphoreType.DMA,
        pltpu.SemaphoreType.DMA,
        pltpu.SemaphoreType.DMA,
        pltpu.SemaphoreType.DMA,
        pltpu.SemaphoreType.DMA,
        pltpu.SemaphoreType.DMA,
    ],
    compiler_params=_sc_params,
)
def _agg2(h_hbm, ei_hbm, out_hbm,
          src_v, dst_v, b0, b1, b2, b3, idx_wb, acc,
          g0, g1, g2, g3, s0, s1, s2, s3):
    bufs = (b0, b1, b2, b3)
    gsems = (g0, g1, g2, g3)
    ssems = (s0, s1, s2, s3)
    cid = lax.axis_index("c")
    sid = lax.axis_index("s")
    wid = sid * NC + cid
    pltpu.sync_copy(ei_hbm.at[0, pl.ds(wid * NCH, NCH)], src_v)
    pltpu.sync_copy(ei_hbm.at[1, pl.ds(wid * NCH, NCH)], dst_v)

    @pl.when(wid == NW - 1)
    def _():
        _fix_pad_rows(src_v, NCH - (EC - ECR), NCH, (NW - 1) * NCH, False)
        _fix_pad_rows(dst_v, NCH - (EC - ECR), NCH, (NW - 1) * NCH, True)

    _fill_wb_idx(idx_wb, cid, sid)
    _zero_acc(b0, acc, sid, D2)
    plsc.subcore_barrier()
    _ring_loop(h_hbm, acc, src_v, dst_v, bufs, gsems, ssems, NCH)
    plsc.subcore_barrier()
    _writeback(b0, acc, out_hbm, idx_wb, sid)


_B = 1024
_G = pl.cdiv(N, _B)


def _rowblock(i):
    return (i, 0)


def _cblock(i):
    return (0, i, 0)


def _tc1_body(cnt, x, w, o):
    dinv = lax.rsqrt(cnt[...] + 1.0)
    o[...] = jnp.dot(x[...], w[...], preferred_element_type=jnp.float32) * dinv


_tc1 = pl.pallas_call(
    _tc1_body,
    grid=(_G,),
    in_specs=[
        pl.BlockSpec((_B, D1), _rowblock),
        pl.BlockSpec((_B, D0), _rowblock),
        pl.BlockSpec((D0, D1), lambda i: (0, 0)),
    ],
    out_specs=pl.BlockSpec((_B, D1), _rowblock),
    out_shape=jax.ShapeDtypeStruct((N, D1), jnp.float32),
)


def _tc2_body(cnt, p, hp, b, w, o):
    dinv = lax.rsqrt(cnt[...] + 1.0)
    y = jnp.maximum(dinv * (p[...] + hp[...]) + b[...], 0.0)
    o[...] = jnp.dot(y, w[...], preferred_element_type=jnp.float32) * dinv[:, :D2]


_tc2 = pl.pallas_call(
    _tc2_body,
    grid=(_G,),
    in_specs=[
        pl.BlockSpec((_B, D1), _rowblock),
        pl.BlockSpec((_B, D1), _rowblock),
        pl.BlockSpec((_B, D1), _rowblock),
        pl.BlockSpec((1, D1), lambda i: (0, 0)),
        pl.BlockSpec((D1, D2), lambda i: (0, 0)),
    ],
    out_specs=pl.BlockSpec((_B, D2), _rowblock),
    out_shape=jax.ShapeDtypeStruct((N, D2), jnp.float32),
)


def _tc3_body(cnt, q, hp, b, o):
    dinv = lax.rsqrt(cnt[...] + 1.0)
    o[...] = dinv[:, :D2] * (q[:, :D2] + q[:, D2:] + hp[...]) + b[...]


_tc3 = pl.pallas_call(
    _tc3_body,
    grid=(_G,),
    in_specs=[
        pl.BlockSpec((_B, D1), _rowblock),
        pl.BlockSpec((_B, 2 * D2), _rowblock),
        pl.BlockSpec((_B, D2), _rowblock),
        pl.BlockSpec((1, D2), lambda i: (0, 0)),
    ],
    out_specs=pl.BlockSpec((_B, D2), _rowblock),
    out_shape=jax.ShapeDtypeStruct((N, D2), jnp.float32),
)


def kernel(x, edge_index, W1, b1, W2, b2):
    ei3 = _prep(edge_index)
    cnt = _hist(ei3)
    h1p = _tc1(cnt, x, W1)
    h2n = h1p.reshape(2 * N, FH)
    p = _agg1(h2n, ei3).reshape(NPAD, D1)
    h2p = _tc2(cnt, p, h1p, b1.reshape(1, D1), W2)
    q = _agg2(h2p, ei3).reshape(NPAD, 2 * D2)
    out = _tc3(cnt, q, h2p, b2.reshape(1, D2))
    return out


# speedup vs baseline: 41.0047x; 1.0446x over previous
import functools

import jax
import jax.numpy as jnp
from jax import lax
from jax.experimental import pallas as pl
from jax.experimental.pallas import tpu as pltpu
from jax.experimental.pallas import tpu_sc as plsc

N = 10000
E = 320000
D0, D1, D2 = 128, 128, 64
FH = D1 // 2

NC = 2
NS = 16
NW = NC * NS
CH = 128
NCH = 80
NCH2 = 160
EC = NW * NCH
ECR = E // CH
NPAD = 10240
RPT = NPAD // NS
KB = RPT // CH
K = 4

_mesh = plsc.VectorSubcoreMesh(core_axis_name="c", subcore_axis_name="s")
_sc_params = pltpu.CompilerParams(use_tc_tiling_on_sc=False)

_PEB = 256


def _prep_body(ei, o):
    o[...] = ei[...].reshape(2, _PEB, CH)


_prep = pl.pallas_call(
    _prep_body,
    grid=(EC // _PEB,),
    in_specs=[pl.BlockSpec((2, _PEB * CH), lambda c: (0, c))],
    out_specs=pl.BlockSpec((2, _PEB, CH), lambda c: (0, c, 0)),
    out_shape=jax.ShapeDtypeStruct((2, EC, CH), jnp.int32),
)


def _xform_src(src_v, cid, nch):
    def xform(r, carry):
        for c in range(CH // 16):
            v = src_v[r, pl.ds(c * 16, 16)]
            src_v[r, pl.ds(c * 16, 16)] = v * 2 + cid
        return carry

    lax.fori_loop(0, nch, xform, 0)


def _fix_pad_rows(idx_v, lo, hi, base_row, is_dst):
    for r in range(lo, hi):
        for i in range(CH // 16):
            k = (base_row + r) * CH + i * 16 + lax.iota(jnp.int32, 16)
            if is_dst:
                idx_v[r, pl.ds(i * 16, 16)] = N + k % (NPAD - N)
            else:
                idx_v[r, pl.ds(i * 16, 16)] = k % N


NRC = NPAD // NC
RPT2 = NRC // NS
GB = 80


@functools.partial(
    pl.kernel,
    out_type=jax.ShapeDtypeStruct((NPAD, D1), jnp.float32),
    mesh=_mesh,
    scratch_types=[
        pltpu.VMEM((NCH2, CH), jnp.int32),
        pltpu.VMEM((CH,), jnp.float32),
        pltpu.VMEM((RPT,), jnp.float32),
        pltpu.VMEM((RPT2,), jnp.float32),
        pltpu.VMEM((GB, D1), jnp.float32),
        pltpu.VMEM_SHARED((NPAD,), jnp.float32),
        pltpu.SemaphoreType.DMA,
    ],
    compiler_params=_sc_params,
)
def _hist(ei_hbm, out_hbm, dst_v, ones_v, zero_v, cnt_v, stage_v, acc, sem):
    cid = lax.axis_index("c")
    sid = lax.axis_index("s")
    for i in range(CH // 16):
        ones_v[pl.ds(i * 16, 16)] = jnp.ones((16,), jnp.float32)
    for i in range(RPT // 16):
        zero_v[pl.ds(i * 16, 16)] = jnp.zeros((16,), jnp.float32)
    pltpu.sync_copy(zero_v, acc.at[pl.ds(sid * RPT, RPT)])
    pltpu.sync_copy(ei_hbm.at[1, pl.ds(sid * NCH2, NCH2)], dst_v)

    @pl.when(sid == NS - 1)
    def _():
        _fix_pad_rows(dst_v, NCH2 - (EC - ECR), NCH2, (NS - 1) * NCH2, True)

    plsc.subcore_barrier()

    def body(j, carry):
        pltpu.async_copy(ones_v, acc.at[dst_v.at[j]], sem, add=True)
        return carry

    lax.fori_loop(0, NCH2, body, 0)

    def drain(j, carry):
        pltpu.make_async_copy(ones_v, acc.at[dst_v.at[0]], sem).wait()
        return carry

    lax.fori_loop(0, NCH2, drain, 0)
    plsc.subcore_barrier()
    row0 = cid * NRC + sid * RPT2
    pltpu.sync_copy(acc.at[pl.ds(row0, RPT2)], cnt_v)
    for g in range(RPT2 // GB):
        def brow(r16, carry):
            c16 = cnt_v[pl.ds(g * GB + r16 * 16, 16)]
            for l in range(16):
                row = r16 * 16 + l
                v = jnp.full((16,), c16[l], jnp.float32)
                for i in range(D1 // 16):
                    stage_v[row, pl.ds(i * 16, 16)] = v
            return carry

        lax.fori_loop(0, GB // 16, brow, 0)
        pltpu.sync_copy(stage_v, out_hbm.at[pl.ds(row0 + g * GB, GB)])


def _ring_loop(h_hbm, acc, src_v, dst_v, bufs, gsems, ssems, nch):
    for s in range(K):
        pltpu.async_copy(h_hbm.at[src_v.at[s]], bufs[s], gsems[s])

    def round_body(r, carry):
        j0 = r * K
        for s in range(K):
            j = j0 + s
            pltpu.make_async_copy(h_hbm.at[src_v.at[j]], bufs[s], gsems[s]).wait()
            pltpu.async_copy(bufs[s], acc.at[dst_v.at[j]], ssems[s], add=True)
        for s in range(K):
            jn = j0 + K + s

            @pl.when(jn < nch)
            def _():
                pltpu.make_async_copy(
                    bufs[s], acc.at[dst_v.at[j0 + s]], ssems[s]).wait()
                pltpu.async_copy(h_hbm.at[src_v.at[jn]], bufs[s], gsems[s])
        return carry

    lax.fori_loop(0, nch // K, round_body, 0)
    for s in range(K):
        pltpu.make_async_copy(
            bufs[s], acc.at[dst_v.at[nch - K + s]], ssems[s]).wait()


def _zero_acc(stage_v, acc, sid, fh):
    def zrow(r, carry):
        for c in range(fh // 16):
            stage_v[r, pl.ds(c * 16, 16)] = jnp.zeros((16,), jnp.float32)
        return carry

    lax.fori_loop(0, CH, zrow, 0)
    for k in range(KB):
        pltpu.sync_copy(stage_v, acc.at[pl.ds(sid * RPT + k * CH, CH)])


def _fill_wb_idx(idx_wb, cid, sid):
    for k in range(KB):
        for i in range(CH // 16):
            r = sid * RPT + k * CH + i * 16 + lax.iota(jnp.int32, 16)
            idx_wb[k, pl.ds(i * 16, 16)] = r * 2 + cid


def _writeback(stage_v, acc, out_hbm, idx_wb, sid):
    for k in range(KB):
        pltpu.sync_copy(acc.at[pl.ds(sid * RPT + k * CH, CH)], stage_v)
        pltpu.sync_copy(stage_v, out_hbm.at[idx_wb.at[k]])


@functools.partial(
    pl.kernel,
    out_type=jax.ShapeDtypeStruct((2 * NPAD, FH), jnp.float32),
    mesh=_mesh,
    scratch_types=[
        pltpu.VMEM((NCH2, CH), jnp.int32),
        pltpu.VMEM((NCH2, CH), jnp.int32),
        pltpu.VMEM((CH, FH), jnp.float32),
        pltpu.VMEM((CH, FH), jnp.float32),
        pltpu.VMEM((CH, FH), jnp.float32),
        pltpu.VMEM((CH, FH), jnp.float32),
        pltpu.VMEM((KB, CH), jnp.int32),
        pltpu.VMEM_SHARED((NPAD, FH), jnp.float32),
        pltpu.SemaphoreType.DMA,
        pltpu.SemaphoreType.DMA,
        pltpu.SemaphoreType.DMA,
        pltpu.SemaphoreType.DMA,
        pltpu.SemaphoreType.DMA,
        pltpu.SemaphoreType.DMA,
        pltpu.SemaphoreType.DMA,
        pltpu.SemaphoreType.DMA,
    ],
    compiler_params=_sc_params,
)
def _agg1(h2n_hbm, ei_hbm, out_hbm,
          src_v, dst_v, b0, b1, b2, b3, idx_wb, acc,
          g0, g1, g2, g3, s0, s1, s2, s3):
    bufs = (b0, b1, b2, b3)
    gsems = (g0, g1, g2, g3)
    ssems = (s0, s1, s2, s3)
    cid = lax.axis_index("c")
    sid = lax.axis_index("s")
    pltpu.sync_copy(ei_hbm.at[0, pl.ds(sid * NCH2, NCH2)], src_v)
    pltpu.sync_copy(ei_hbm.at[1, pl.ds(sid * NCH2, NCH2)], dst_v)

    @pl.when(sid == NS - 1)
    def _():
        _fix_pad_rows(src_v, NCH2 - (EC - ECR), NCH2, (NS - 1) * NCH2, False)
        _fix_pad_rows(dst_v, NCH2 - (EC - ECR), NCH2, (NS - 1) * NCH2, True)

    _fill_wb_idx(idx_wb, cid, sid)
    _xform_src(src_v, cid, NCH2)
    _zero_acc(b0, acc, sid, FH)
    plsc.subcore_barrier()
    _ring_loop(h2n_hbm, acc, src_v, dst_v, bufs, gsems, ssems, NCH2)
    plsc.subcore_barrier()
    _writeback(b0, acc, out_hbm, idx_wb, sid)


@functools.partial(
    pl.kernel,
    out_type=jax.ShapeDtypeStruct((2 * NPAD, D2), jnp.float32),
    mesh=_mesh,
    scratch_types=[
        pltpu.VMEM((NCH, CH), jnp.int32),
        pltpu.VMEM((NCH, CH), jnp.int32),
        pltpu.VMEM((CH, D2), jnp.float32),
        pltpu.VMEM((CH, D2), jnp.float32),
        pltpu.VMEM((CH, D2), jnp.float32),
        pltpu.VMEM((CH, D2), jnp.float32),
        pltpu.VMEM((KB, CH), jnp.int32),
        pltpu.VMEM_SHARED((NPAD, D2), jnp.float32),
        pltpu.SemaphoreType.DMA,
        pltpu.SemaphoreType.DMA,
        pltpu.SemaphoreType.DMA,
        pltpu.SemaphoreType.DMA,
        pltpu.SemaphoreType.DMA,
        pltpu.SemaphoreType.DMA,
        pltpu.SemaphoreType.DMA,
        pltpu.SemaphoreType.DMA,
    ],
    compiler_params=_sc_params,
)
def _agg2(h_hbm, ei_hbm, out_hbm,
          src_v, dst_v, b0, b1, b2, b3, idx_wb, acc,
          g0, g1, g2, g3, s0, s1, s2, s3):
    bufs = (b0, b1, b2, b3)
    gsems = (g0, g1, g2, g3)
    ssems = (s0, s1, s2, s3)
    cid = lax.axis_index("c")
    sid = lax.axis_index("s")
    wid = sid * NC + cid
    pltpu.sync_copy(ei_hbm.at[0, pl.ds(wid * NCH, NCH)], src_v)
    pltpu.sync_copy(ei_hbm.at[1, pl.ds(wid * NCH, NCH)], dst_v)

    @pl.when(wid == NW - 1)
    def _():
        _fix_pad_rows(src_v, NCH - (EC - ECR), NCH, (NW - 1) * NCH, False)
        _fix_pad_rows(dst_v, NCH - (EC - ECR), NCH, (NW - 1) * NCH, True)

    _fill_wb_idx(idx_wb, cid, sid)
    _xform_src(src_v, cid, NCH)
    _zero_acc(b0, acc, sid, D2)
    plsc.subcore_barrier()
    _ring_loop(h_hbm, acc, src_v, dst_v, bufs, gsems, ssems, NCH)
    plsc.subcore_barrier()
    _writeback(b0, acc, out_hbm, idx_wb, sid)


_B = 2048
_G = pl.cdiv(N, _B)


def _rowblock(i):
    return (i, 0)


def _tc0_body(x, w, o):
    o[...] = jnp.dot(x[...], w[...], preferred_element_type=jnp.float32)


_tc0 = pl.pallas_call(
    _tc0_body,
    grid=(_G,),
    in_specs=[
        pl.BlockSpec((_B, D0), _rowblock),
        pl.BlockSpec((D0, D1), lambda i: (0, 0)),
    ],
    out_specs=pl.BlockSpec((_B, D1), _rowblock),
    out_shape=jax.ShapeDtypeStruct((N, D1), jnp.float32),
)


def _tc1_body(cnt, xw, o):
    dinv = lax.rsqrt(cnt[...] + 1.0)
    o[...] = xw[...] * dinv


_tc1 = pl.pallas_call(
    _tc1_body,
    grid=(_G,),
    in_specs=[
        pl.BlockSpec((_B, D1), _rowblock),
        pl.BlockSpec((_B, D1), _rowblock),
    ],
    out_specs=pl.BlockSpec((_B, D1), _rowblock),
    out_shape=jax.ShapeDtypeStruct((N, D1), jnp.float32),
)


def _tc2_body(cnt, p, hp, b, w, o):
    dinv = lax.rsqrt(cnt[...] + 1.0)
    y = jnp.maximum(dinv * (p[...] + hp[...]) + b[...], 0.0)
    h2 = jnp.dot(y, w[...], preferred_element_type=jnp.float32) * dinv[:, :D2]
    o[...] = jnp.concatenate([h2, h2], axis=1)


_tc2 = pl.pallas_call(
    _tc2_body,
    grid=(_G,),
    in_specs=[
        pl.BlockSpec((_B, D1), _rowblock),
        pl.BlockSpec((_B, D1), _rowblock),
        pl.BlockSpec((_B, D1), _rowblock),
        pl.BlockSpec((1, D1), lambda i: (0, 0)),
        pl.BlockSpec((D1, D2), lambda i: (0, 0)),
    ],
    out_specs=pl.BlockSpec((_B, 2 * D2), _rowblock),
    out_shape=jax.ShapeDtypeStruct((N, 2 * D2), jnp.float32),
)


def _tc3_body(cnt, q, hp, b, o):
    dinv = lax.rsqrt(cnt[...] + 1.0)
    o[...] = dinv[:, :D2] * (q[:, :D2] + q[:, D2:] + hp[:, :D2]) + b[...]


_tc3 = pl.pallas_call(
    _tc3_body,
    grid=(_G,),
    in_specs=[
        pl.BlockSpec((_B, D1), _rowblock),
        pl.BlockSpec((_B, 2 * D2), _rowblock),
        pl.BlockSpec((_B, 2 * D2), _rowblock),
        pl.BlockSpec((1, D2), lambda i: (0, 0)),
    ],
    out_specs=pl.BlockSpec((_B, D2), _rowblock),
    out_shape=jax.ShapeDtypeStruct((N, D2), jnp.float32),
)


def kernel(x, edge_index, W1, b1, W2, b2):
    ei3 = _prep(edge_index)
    xw = _tc0(x, W1)
    cnt = _hist(ei3)
    h1p = _tc1(cnt, xw)
    h2n = h1p.reshape(2 * N, FH)
    p = _agg1(h2n, ei3).reshape(NPAD, D1)
    h2d = _tc2(cnt, p, h1p, b1.reshape(1, D1), W2)
    q = _agg2(h2d.reshape(2 * N, D2), ei3).reshape(NPAD, 2 * D2)
    out = _tc3(cnt, q, h2d, b2.reshape(1, D2))
    return out


# speedup vs baseline: 46.4559x; 1.1329x over previous
import functools

import jax
import jax.numpy as jnp
from jax import lax
from jax.experimental import pallas as pl
from jax.experimental.pallas import tpu as pltpu
from jax.experimental.pallas import tpu_sc as plsc

N = 10000
E = 320000
D0, D1, D2 = 128, 128, 64
FH = D1 // 2

NC = 2
NS = 16
NW = NC * NS
CH = 128
NCH = 80
NCH2 = 160
EC = NW * NCH
ECR = E // CH
NPAD = 10240
RPT = NPAD // NS
KB = RPT // CH
K = 4

_mesh = plsc.VectorSubcoreMesh(core_axis_name="c", subcore_axis_name="s")
_sc_params = pltpu.CompilerParams(use_tc_tiling_on_sc=False)

_PEB = 256


def _prep_body(ei, o):
    o[...] = ei[...].reshape(2, _PEB, CH)


_prep = pl.pallas_call(
    _prep_body,
    grid=(EC // _PEB,),
    in_specs=[pl.BlockSpec((2, _PEB * CH), lambda c: (0, c))],
    out_specs=pl.BlockSpec((2, _PEB, CH), lambda c: (0, c, 0)),
    out_shape=jax.ShapeDtypeStruct((2, EC, CH), jnp.int32),
)


def _xform_src(src_v, cid, nch):
    def xform(r, carry):
        for c in range(CH // 16):
            v = src_v[r, pl.ds(c * 16, 16)]
            src_v[r, pl.ds(c * 16, 16)] = v * 2 + cid
        return carry

    lax.fori_loop(0, nch, xform, 0)


def _fix_pad_rows(idx_v, lo, hi, base_row, is_dst):
    for r in range(lo, hi):
        for i in range(CH // 16):
            k = (base_row + r) * CH + i * 16 + lax.iota(jnp.int32, 16)
            if is_dst:
                idx_v[r, pl.ds(i * 16, 16)] = N + k % (NPAD - N)
            else:
                idx_v[r, pl.ds(i * 16, 16)] = k % N


NRC = NPAD // NC
RPT2 = NRC // NS
GB = 80


@functools.partial(
    pl.kernel,
    out_type=jax.ShapeDtypeStruct((NPAD, D1), jnp.float32),
    mesh=_mesh,
    scratch_types=[
        pltpu.VMEM((NCH2, CH), jnp.int32),
        pltpu.VMEM((CH,), jnp.float32),
        pltpu.VMEM((RPT,), jnp.float32),
        pltpu.VMEM((RPT2,), jnp.float32),
        pltpu.VMEM((GB, D1), jnp.float32),
        pltpu.VMEM_SHARED((NPAD,), jnp.float32),
        pltpu.SemaphoreType.DMA,
    ],
    compiler_params=_sc_params,
)
def _hist(ei_hbm, out_hbm, dst_v, ones_v, zero_v, cnt_v, stage_v, acc, sem):
    cid = lax.axis_index("c")
    sid = lax.axis_index("s")
    for i in range(CH // 16):
        ones_v[pl.ds(i * 16, 16)] = jnp.ones((16,), jnp.float32)
    for i in range(RPT // 16):
        zero_v[pl.ds(i * 16, 16)] = jnp.zeros((16,), jnp.float32)
    pltpu.sync_copy(zero_v, acc.at[pl.ds(sid * RPT, RPT)])
    pltpu.sync_copy(ei_hbm.at[1, pl.ds(sid * NCH2, NCH2)], dst_v)

    @pl.when(sid == NS - 1)
    def _():
        _fix_pad_rows(dst_v, NCH2 - (EC - ECR), NCH2, (NS - 1) * NCH2, True)

    plsc.subcore_barrier()

    def body(j, carry):
        pltpu.async_copy(ones_v, acc.at[dst_v.at[j]], sem, add=True)
        return carry

    lax.fori_loop(0, NCH2, body, 0)

    def drain(j, carry):
        pltpu.make_async_copy(ones_v, acc.at[dst_v.at[0]], sem).wait()
        return carry

    lax.fori_loop(0, NCH2, drain, 0)
    plsc.subcore_barrier()
    row0 = cid * NRC + sid * RPT2
    pltpu.sync_copy(acc.at[pl.ds(row0, RPT2)], cnt_v)
    for g in range(RPT2 // GB):
        def brow(r16, carry):
            c16 = cnt_v[pl.ds(g * GB + r16 * 16, 16)]
            for l in range(16):
                row = r16 * 16 + l
                v = jnp.full((16,), c16[l], jnp.float32)
                for i in range(D1 // 16):
                    stage_v[row, pl.ds(i * 16, 16)] = v
            return carry

        lax.fori_loop(0, GB // 16, brow, 0)
        pltpu.sync_copy(stage_v, out_hbm.at[pl.ds(row0 + g * GB, GB)])


def _ring_loop(h_hbm, acc, src_v, dst_v, bufs, gsems, ssems, nch):
    for s in range(K):
        pltpu.async_copy(h_hbm.at[src_v.at[s]], bufs[s], gsems[s])

    def round_body(r, carry):
        j0 = r * K
        for s in range(K):
            j = j0 + s
            pltpu.make_async_copy(h_hbm.at[src_v.at[j]], bufs[s], gsems[s]).wait()
            pltpu.async_copy(bufs[s], acc.at[dst_v.at[j]], ssems[s], add=True)
        for s in range(K):
            jn = j0 + K + s

            @pl.when(jn < nch)
            def _():
                pltpu.make_async_copy(
                    bufs[s], acc.at[dst_v.at[j0 + s]], ssems[s]).wait()
                pltpu.async_copy(h_hbm.at[src_v.at[jn]], bufs[s], gsems[s])
        return carry

    lax.fori_loop(0, nch // K, round_body, 0)
    for s in range(K):
        pltpu.make_async_copy(
            bufs[s], acc.at[dst_v.at[nch - K + s]], ssems[s]).wait()


def _zero_acc(stage_v, acc, sid, fh):
    def zrow(r, carry):
        for c in range(fh // 32):
            stage_v[r, pl.ds(c * 32, 32)] = jnp.zeros((32,), jnp.bfloat16)
        return carry

    lax.fori_loop(0, CH, zrow, 0)
    for k in range(KB):
        pltpu.sync_copy(stage_v, acc.at[pl.ds(sid * RPT + k * CH, CH)])


def _fill_wb_idx(idx_wb, cid, sid):
    for k in range(KB):
        for i in range(CH // 16):
            r = sid * RPT + k * CH + i * 16 + lax.iota(jnp.int32, 16)
            idx_wb[k, pl.ds(i * 16, 16)] = r * 2 + cid


def _writeback(stage_v, acc, out_hbm, idx_wb, sid):
    for k in range(KB):
        pltpu.sync_copy(acc.at[pl.ds(sid * RPT + k * CH, CH)], stage_v)
        pltpu.sync_copy(stage_v, out_hbm.at[idx_wb.at[k]])


@functools.partial(
    pl.kernel,
    out_type=jax.ShapeDtypeStruct((2 * NPAD, FH), jnp.bfloat16),
    mesh=_mesh,
    scratch_types=[
        pltpu.VMEM((NCH2, CH), jnp.int32),
        pltpu.VMEM((NCH2, CH), jnp.int32),
        pltpu.VMEM((CH, FH), jnp.bfloat16),
        pltpu.VMEM((CH, FH), jnp.bfloat16),
        pltpu.VMEM((CH, FH), jnp.bfloat16),
        pltpu.VMEM((CH, FH), jnp.bfloat16),
        pltpu.VMEM((KB, CH), jnp.int32),
        pltpu.VMEM_SHARED((NPAD, FH), jnp.bfloat16),
        pltpu.SemaphoreType.DMA,
        pltpu.SemaphoreType.DMA,
        pltpu.SemaphoreType.DMA,
        pltpu.SemaphoreType.DMA,
        pltpu.SemaphoreType.DMA,
        pltpu.SemaphoreType.DMA,
        pltpu.SemaphoreType.DMA,
        pltpu.SemaphoreType.DMA,
    ],
    compiler_params=_sc_params,
)
def _agg1(h2n_hbm, ei_hbm, out_hbm,
          src_v, dst_v, b0, b1, b2, b3, idx_wb, acc,
          g0, g1, g2, g3, s0, s1, s2, s3):
    bufs = (b0, b1, b2, b3)
    gsems = (g0, g1, g2, g3)
    ssems = (s0, s1, s2, s3)
    cid = lax.axis_index("c")
    sid = lax.axis_index("s")
    pltpu.sync_copy(ei_hbm.at[0, pl.ds(sid * NCH2, NCH2)], src_v)
    pltpu.sync_copy(ei_hbm.at[1, pl.ds(sid * NCH2, NCH2)], dst_v)

    @pl.when(sid == NS - 1)
    def _():
        _fix_pad_rows(src_v, NCH2 - (EC - ECR), NCH2, (NS - 1) * NCH2, False)
        _fix_pad_rows(dst_v, NCH2 - (EC - ECR), NCH2, (NS - 1) * NCH2, True)

    _fill_wb_idx(idx_wb, cid, sid)
    _xform_src(src_v, cid, NCH2)
    _zero_acc(b0, acc, sid, FH)
    plsc.subcore_barrier()
    _ring_loop(h2n_hbm, acc, src_v, dst_v, bufs, gsems, ssems, NCH2)
    plsc.subcore_barrier()
    _writeback(b0, acc, out_hbm, idx_wb, sid)


@functools.partial(
    pl.kernel,
    out_type=jax.ShapeDtypeStruct((2 * NPAD, D2), jnp.bfloat16),
    mesh=_mesh,
    scratch_types=[
        pltpu.VMEM((NCH, CH), jnp.int32),
        pltpu.VMEM((NCH, CH), jnp.int32),
        pltpu.VMEM((CH, D2), jnp.bfloat16),
        pltpu.VMEM((CH, D2), jnp.bfloat16),
        pltpu.VMEM((CH, D2), jnp.bfloat16),
        pltpu.VMEM((CH, D2), jnp.bfloat16),
        pltpu.VMEM((KB, CH), jnp.int32),
        pltpu.VMEM_SHARED((NPAD, D2), jnp.bfloat16),
        pltpu.SemaphoreType.DMA,
        pltpu.SemaphoreType.DMA,
        pltpu.SemaphoreType.DMA,
        pltpu.SemaphoreType.DMA,
        pltpu.SemaphoreType.DMA,
        pltpu.SemaphoreType.DMA,
        pltpu.SemaphoreType.DMA,
        pltpu.SemaphoreType.DMA,
    ],
    compiler_params=_sc_params,
)
def _agg2(h_hbm, ei_hbm, out_hbm,
          src_v, dst_v, b0, b1, b2, b3, idx_wb, acc,
          g0, g1, g2, g3, s0, s1, s2, s3):
    bufs = (b0, b1, b2, b3)
    gsems = (g0, g1, g2, g3)
    ssems = (s0, s1, s2, s3)
    cid = lax.axis_index("c")
    sid = lax.axis_index("s")
    wid = sid * NC + cid
    pltpu.sync_copy(ei_hbm.at[0, pl.ds(wid * NCH, NCH)], src_v)
    pltpu.sync_copy(ei_hbm.at[1, pl.ds(wid * NCH, NCH)], dst_v)

    @pl.when(wid == NW - 1)
    def _():
        _fix_pad_rows(src_v, NCH - (EC - ECR), NCH, (NW - 1) * NCH, False)
        _fix_pad_rows(dst_v, NCH - (EC - ECR), NCH, (NW - 1) * NCH, True)

    _fill_wb_idx(idx_wb, cid, sid)
    _xform_src(src_v, cid, NCH)
    _zero_acc(b0, acc, sid, D2)
    plsc.subcore_barrier()
    _ring_loop(h_hbm, acc, src_v, dst_v, bufs, gsems, ssems, NCH)
    plsc.subcore_barrier()
    _writeback(b0, acc, out_hbm, idx_wb, sid)


_B = 2048
_G = pl.cdiv(N, _B)


def _rowblock(i):
    return (i, 0)


def _tc0_body(x, w, o):
    o[...] = jnp.dot(x[...], w[...], preferred_element_type=jnp.float32)


_tc0 = pl.pallas_call(
    _tc0_body,
    grid=(_G,),
    in_specs=[
        pl.BlockSpec((_B, D0), _rowblock),
        pl.BlockSpec((D0, D1), lambda i: (0, 0)),
    ],
    out_specs=pl.BlockSpec((_B, D1), _rowblock),
    out_shape=jax.ShapeDtypeStruct((N, D1), jnp.float32),
)


def _tc1_body(cnt, xw, o):
    dinv = lax.rsqrt(cnt[...] + 1.0)
    o[...] = (xw[...] * dinv).astype(jnp.bfloat16)


_tc1 = pl.pallas_call(
    _tc1_body,
    grid=(_G,),
    in_specs=[
        pl.BlockSpec((_B, D1), _rowblock),
        pl.BlockSpec((_B, D1), _rowblock),
    ],
    out_specs=pl.BlockSpec((_B, D1), _rowblock),
    out_shape=jax.ShapeDtypeStruct((N, D1), jnp.bfloat16),
)


def _tc2_body(cnt, p, hp, b, w, o):
    dinv = lax.rsqrt(cnt[...] + 1.0)
    s = p[...].astype(jnp.float32) + hp[...].astype(jnp.float32)
    y = jnp.maximum(dinv * s + b[...], 0.0)
    h2 = jnp.dot(y, w[...], preferred_element_type=jnp.float32) * dinv[:, :D2]
    o[...] = jnp.concatenate([h2, h2], axis=1).astype(jnp.bfloat16)


_tc2 = pl.pallas_call(
    _tc2_body,
    grid=(_G,),
    in_specs=[
        pl.BlockSpec((_B, D1), _rowblock),
        pl.BlockSpec((_B, D1), _rowblock),
        pl.BlockSpec((_B, D1), _rowblock),
        pl.BlockSpec((1, D1), lambda i: (0, 0)),
        pl.BlockSpec((D1, D2), lambda i: (0, 0)),
    ],
    out_specs=pl.BlockSpec((_B, 2 * D2), _rowblock),
    out_shape=jax.ShapeDtypeStruct((N, 2 * D2), jnp.bfloat16),
)


def _tc3_body(cnt, q, hp, b, o):
    dinv = lax.rsqrt(cnt[...] + 1.0)
    qf = q[...].astype(jnp.float32)
    hpf = hp[...].astype(jnp.float32)
    o[...] = dinv[:, :D2] * (qf[:, :D2] + qf[:, D2:] + hpf[:, :D2]) + b[...]


_tc3 = pl.pallas_call(
    _tc3_body,
    grid=(_G,),
    in_specs=[
        pl.BlockSpec((_B, D1), _rowblock),
        pl.BlockSpec((_B, 2 * D2), _rowblock),
        pl.BlockSpec((_B, 2 * D2), _rowblock),
        pl.BlockSpec((1, D2), lambda i: (0, 0)),
    ],
    out_specs=pl.BlockSpec((_B, D2), _rowblock),
    out_shape=jax.ShapeDtypeStruct((N, D2), jnp.float32),
)


def kernel(x, edge_index, W1, b1, W2, b2):
    ei3 = _prep(edge_index)
    xw = _tc0(x, W1)
    cnt = _hist(ei3)
    h1p = _tc1(cnt, xw)
    h2n = h1p.reshape(2 * N, FH)
    p = _agg1(h2n, ei3).reshape(NPAD, D1)
    h2d = _tc2(cnt, p, h1p, b1.reshape(1, D1), W2)
    q = _agg2(h2d.reshape(2 * N, D2), ei3).reshape(NPAD, 2 * D2)
    out = _tc3(cnt, q, h2d, b2.reshape(1, D2))
    return out


# speedup vs baseline: 49.2618x; 1.0604x over previous
import functools

import jax
import jax.numpy as jnp
from jax import lax
from jax.experimental import pallas as pl
from jax.experimental.pallas import tpu as pltpu
from jax.experimental.pallas import tpu_sc as plsc

N = 10000
E = 320000
D0, D1, D2 = 128, 128, 64
FH = D1 // 2

NC = 2
NS = 16
NW = NC * NS
CH = 128
NCH = 80
NCH2 = 160
EC = NW * NCH
ECR = E // CH
NPAD = 10240
RPT = NPAD // NS
KB = RPT // CH
K = 8

_mesh = plsc.VectorSubcoreMesh(core_axis_name="c", subcore_axis_name="s")
_sc_params = pltpu.CompilerParams(use_tc_tiling_on_sc=False)

_PEB = 256


def _prep_body(ei, o):
    o[...] = ei[...].reshape(2, _PEB, CH)


_prep = pl.pallas_call(
    _prep_body,
    grid=(EC // _PEB,),
    in_specs=[pl.BlockSpec((2, _PEB * CH), lambda c: (0, c))],
    out_specs=pl.BlockSpec((2, _PEB, CH), lambda c: (0, c, 0)),
    out_shape=jax.ShapeDtypeStruct((2, EC, CH), jnp.int32),
)


def _xform_src(src_v, cid, nch):
    def xform(r, carry):
        for c in range(CH // 16):
            v = src_v[r, pl.ds(c * 16, 16)]
            src_v[r, pl.ds(c * 16, 16)] = v * 2 + cid
        return carry

    lax.fori_loop(0, nch, xform, 0)


def _fix_pad_rows(idx_v, lo, hi, base_row, is_dst):
    for r in range(lo, hi):
        for i in range(CH // 16):
            k = (base_row + r) * CH + i * 16 + lax.iota(jnp.int32, 16)
            if is_dst:
                idx_v[r, pl.ds(i * 16, 16)] = N + k % (NPAD - N)
            else:
                idx_v[r, pl.ds(i * 16, 16)] = k % N


NRC = NPAD // NC
RPT2 = NRC // NS
GB = 80


@functools.partial(
    pl.kernel,
    out_type=jax.ShapeDtypeStruct((NPAD, D1), jnp.float32),
    mesh=_mesh,
    scratch_types=[
        pltpu.VMEM((NCH2, CH), jnp.int32),
        pltpu.VMEM((CH,), jnp.float32),
        pltpu.VMEM((RPT,), jnp.float32),
        pltpu.VMEM((RPT2,), jnp.float32),
        pltpu.VMEM((GB, D1), jnp.float32),
        pltpu.VMEM_SHARED((NPAD,), jnp.float32),
        pltpu.SemaphoreType.DMA,
    ],
    compiler_params=_sc_params,
)
def _hist(ei_hbm, out_hbm, dst_v, ones_v, zero_v, cnt_v, stage_v, acc, sem):
    cid = lax.axis_index("c")
    sid = lax.axis_index("s")
    for i in range(CH // 16):
        ones_v[pl.ds(i * 16, 16)] = jnp.ones((16,), jnp.float32)
    for i in range(RPT // 16):
        zero_v[pl.ds(i * 16, 16)] = jnp.zeros((16,), jnp.float32)
    pltpu.sync_copy(zero_v, acc.at[pl.ds(sid * RPT, RPT)])
    pltpu.sync_copy(ei_hbm.at[1, pl.ds(sid * NCH2, NCH2)], dst_v)

    @pl.when(sid == NS - 1)
    def _():
        _fix_pad_rows(dst_v, NCH2 - (EC - ECR), NCH2, (NS - 1) * NCH2, True)

    plsc.subcore_barrier()

    def body(j, carry):
        pltpu.async_copy(ones_v, acc.at[dst_v.at[j]], sem, add=True)
        return carry

    lax.fori_loop(0, NCH2, body, 0)

    def drain(j, carry):
        pltpu.make_async_copy(ones_v, acc.at[dst_v.at[0]], sem).wait()
        return carry

    lax.fori_loop(0, NCH2, drain, 0)
    plsc.subcore_barrier()
    row0 = cid * NRC + sid * RPT2
    pltpu.sync_copy(acc.at[pl.ds(row0, RPT2)], cnt_v)
    for g in range(RPT2 // GB):
        def brow(r16, carry):
            c16 = cnt_v[pl.ds(g * GB + r16 * 16, 16)]
            for l in range(16):
                row = r16 * 16 + l
                v = jnp.full((16,), c16[l], jnp.float32)
                for i in range(D1 // 16):
                    stage_v[row, pl.ds(i * 16, 16)] = v
            return carry

        lax.fori_loop(0, GB // 16, brow, 0)
        pltpu.sync_copy(stage_v, out_hbm.at[pl.ds(row0 + g * GB, GB)])


def _ring_loop(h_hbm, acc, src_v, dst_v, bufs, gsems, ssems, nch):
    for s in range(K):
        pltpu.async_copy(h_hbm.at[src_v.at[s]], bufs[s], gsems[s])

    def round_body(r, carry):
        j0 = r * K
        for s in range(K):
            j = j0 + s
            pltpu.make_async_copy(h_hbm.at[src_v.at[j]], bufs[s], gsems[s]).wait()
            pltpu.async_copy(bufs[s], acc.at[dst_v.at[j]], ssems[s], add=True)
        for s in range(K):
            jn = j0 + K + s

            @pl.when(jn < nch)
            def _():
                pltpu.make_async_copy(
                    bufs[s], acc.at[dst_v.at[j0 + s]], ssems[s]).wait()
                pltpu.async_copy(h_hbm.at[src_v.at[jn]], bufs[s], gsems[s])
        return carry

    lax.fori_loop(0, nch // K, round_body, 0)
    for s in range(K):
        pltpu.make_async_copy(
            bufs[s], acc.at[dst_v.at[nch - K + s]], ssems[s]).wait()


def _zero_acc(stage_v, acc, sid, fh):
    def zrow(r, carry):
        for c in range(fh // 32):
            stage_v[r, pl.ds(c * 32, 32)] = jnp.zeros((32,), jnp.bfloat16)
        return carry

    lax.fori_loop(0, CH, zrow, 0)
    for k in range(KB):
        pltpu.sync_copy(stage_v, acc.at[pl.ds(sid * RPT + k * CH, CH)])


def _fill_wb_idx(idx_wb, cid, sid):
    for k in range(KB):
        for i in range(CH // 16):
            r = sid * RPT + k * CH + i * 16 + lax.iota(jnp.int32, 16)
            idx_wb[k, pl.ds(i * 16, 16)] = r * 2 + cid


def _writeback(stage_v, acc, out_hbm, idx_wb, sid):
    for k in range(KB):
        pltpu.sync_copy(acc.at[pl.ds(sid * RPT + k * CH, CH)], stage_v)
        pltpu.sync_copy(stage_v, out_hbm.at[idx_wb.at[k]])


@functools.partial(
    pl.kernel,
    out_type=jax.ShapeDtypeStruct((2 * NPAD, FH), jnp.bfloat16),
    mesh=_mesh,
    scratch_types=[
        pltpu.VMEM((NCH2, CH), jnp.int32),
        pltpu.VMEM((NCH2, CH), jnp.int32),
        pltpu.VMEM((CH, FH), jnp.bfloat16),
        pltpu.VMEM((CH, FH), jnp.bfloat16),
        pltpu.VMEM((CH, FH), jnp.bfloat16),
        pltpu.VMEM((CH, FH), jnp.bfloat16),
        pltpu.VMEM((CH, FH), jnp.bfloat16),
        pltpu.VMEM((CH, FH), jnp.bfloat16),
        pltpu.VMEM((CH, FH), jnp.bfloat16),
        pltpu.VMEM((CH, FH), jnp.bfloat16),
        pltpu.VMEM((KB, CH), jnp.int32),
        pltpu.VMEM_SHARED((NPAD, FH), jnp.bfloat16),
    ] + [pltpu.SemaphoreType.DMA] * (2 * K),
    compiler_params=_sc_params,
)
def _agg1(h2n_hbm, ei_hbm, out_hbm,
          src_v, dst_v, b0, b1, b2, b3, b4, b5, b6, b7, idx_wb, acc,
          *sems):
    bufs = (b0, b1, b2, b3, b4, b5, b6, b7)
    gsems = sems[:K]
    ssems = sems[K:]
    cid = lax.axis_index("c")
    sid = lax.axis_index("s")
    pltpu.sync_copy(ei_hbm.at[0, pl.ds(sid * NCH2, NCH2)], src_v)
    pltpu.sync_copy(ei_hbm.at[1, pl.ds(sid * NCH2, NCH2)], dst_v)

    @pl.when(sid == NS - 1)
    def _():
        _fix_pad_rows(src_v, NCH2 - (EC - ECR), NCH2, (NS - 1) * NCH2, False)
        _fix_pad_rows(dst_v, NCH2 - (EC - ECR), NCH2, (NS - 1) * NCH2, True)

    _fill_wb_idx(idx_wb, cid, sid)
    _xform_src(src_v, cid, NCH2)
    _zero_acc(b0, acc, sid, FH)
    plsc.subcore_barrier()
    _ring_loop(h2n_hbm, acc, src_v, dst_v, bufs, gsems, ssems, NCH2)
    plsc.subcore_barrier()
    _writeback(b0, acc, out_hbm, idx_wb, sid)


@functools.partial(
    pl.kernel,
    out_type=jax.ShapeDtypeStruct((2 * NPAD, D2), jnp.bfloat16),
    mesh=_mesh,
    scratch_types=[
        pltpu.VMEM((NCH, CH), jnp.int32),
        pltpu.VMEM((NCH, CH), jnp.int32),
        pltpu.VMEM((CH, D2), jnp.bfloat16),
        pltpu.VMEM((CH, D2), jnp.bfloat16),
        pltpu.VMEM((CH, D2), jnp.bfloat16),
        pltpu.VMEM((CH, D2), jnp.bfloat16),
        pltpu.VMEM((CH, D2), jnp.bfloat16),
        pltpu.VMEM((CH, D2), jnp.bfloat16),
        pltpu.VMEM((CH, D2), jnp.bfloat16),
        pltpu.VMEM((CH, D2), jnp.bfloat16),
        pltpu.VMEM((KB, CH), jnp.int32),
        pltpu.VMEM_SHARED((NPAD, D2), jnp.bfloat16),
    ] + [pltpu.SemaphoreType.DMA] * (2 * K),
    compiler_params=_sc_params,
)
def _agg2(h_hbm, ei_hbm, out_hbm,
          src_v, dst_v, b0, b1, b2, b3, b4, b5, b6, b7, idx_wb, acc,
          *sems):
    bufs = (b0, b1, b2, b3, b4, b5, b6, b7)
    gsems = sems[:K]
    ssems = sems[K:]
    cid = lax.axis_index("c")
    sid = lax.axis_index("s")
    wid = sid * NC + cid
    pltpu.sync_copy(ei_hbm.at[0, pl.ds(wid * NCH, NCH)], src_v)
    pltpu.sync_copy(ei_hbm.at[1, pl.ds(wid * NCH, NCH)], dst_v)

    @pl.when(wid == NW - 1)
    def _():
        _fix_pad_rows(src_v, NCH - (EC - ECR), NCH, (NW - 1) * NCH, False)
        _fix_pad_rows(dst_v, NCH - (EC - ECR), NCH, (NW - 1) * NCH, True)

    _fill_wb_idx(idx_wb, cid, sid)
    _xform_src(src_v, cid, NCH)
    _zero_acc(b0, acc, sid, D2)
    plsc.subcore_barrier()
    _ring_loop(h_hbm, acc, src_v, dst_v, bufs, gsems, ssems, NCH)
    plsc.subcore_barrier()
    _writeback(b0, acc, out_hbm, idx_wb, sid)


_B = 2048
_G = pl.cdiv(N, _B)


def _rowblock(i):
    return (i, 0)


def _tc0_body(x, w, o):
    o[...] = jnp.dot(x[...], w[...], preferred_element_type=jnp.float32)


_tc0 = pl.pallas_call(
    _tc0_body,
    grid=(_G,),
    in_specs=[
        pl.BlockSpec((_B, D0), _rowblock),
        pl.BlockSpec((D0, D1), lambda i: (0, 0)),
    ],
    out_specs=pl.BlockSpec((_B, D1), _rowblock),
    out_shape=jax.ShapeDtypeStruct((N, D1), jnp.float32),
)


def _tc1_body(cnt, xw, o):
    dinv = lax.rsqrt(cnt[...] + 1.0)
    o[...] = (xw[...] * dinv).astype(jnp.bfloat16)


_tc1 = pl.pallas_call(
    _tc1_body,
    grid=(_G,),
    in_specs=[
        pl.BlockSpec((_B, D1), _rowblock),
        pl.BlockSpec((_B, D1), _rowblock),
    ],
    out_specs=pl.BlockSpec((_B, D1), _rowblock),
    out_shape=jax.ShapeDtypeStruct((N, D1), jnp.bfloat16),
)


def _tc2_body(cnt, p, hp, b, w, o):
    dinv = lax.rsqrt(cnt[...] + 1.0)
    s = p[...].astype(jnp.float32) + hp[...].astype(jnp.float32)
    y = jnp.maximum(dinv * s + b[...], 0.0)
    h2 = jnp.dot(y, w[...], preferred_element_type=jnp.float32) * dinv[:, :D2]
    o[...] = jnp.concatenate([h2, h2], axis=1).astype(jnp.bfloat16)


_tc2 = pl.pallas_call(
    _tc2_body,
    grid=(_G,),
    in_specs=[
        pl.BlockSpec((_B, D1), _rowblock),
        pl.BlockSpec((_B, D1), _rowblock),
        pl.BlockSpec((_B, D1), _rowblock),
        pl.BlockSpec((1, D1), lambda i: (0, 0)),
        pl.BlockSpec((D1, D2), lambda i: (0, 0)),
    ],
    out_specs=pl.BlockSpec((_B, 2 * D2), _rowblock),
    out_shape=jax.ShapeDtypeStruct((N, 2 * D2), jnp.bfloat16),
)


def _tc3_body(cnt, q, hp, b, o):
    dinv = lax.rsqrt(cnt[...] + 1.0)
    qf = q[...].astype(jnp.float32)
    hpf = hp[...].astype(jnp.float32)
    o[...] = dinv[:, :D2] * (qf[:, :D2] + qf[:, D2:] + hpf[:, :D2]) + b[...]


_tc3 = pl.pallas_call(
    _tc3_body,
    grid=(_G,),
    in_specs=[
        pl.BlockSpec((_B, D1), _rowblock),
        pl.BlockSpec((_B, 2 * D2), _rowblock),
        pl.BlockSpec((_B, 2 * D2), _rowblock),
        pl.BlockSpec((1, D2), lambda i: (0, 0)),
    ],
    out_specs=pl.BlockSpec((_B, D2), _rowblock),
    out_shape=jax.ShapeDtypeStruct((N, D2), jnp.float32),
)


def kernel(x, edge_index, W1, b1, W2, b2):
    ei3 = _prep(edge_index)
    xw = _tc0(x, W1)
    cnt = _hist(ei3)
    h1p = _tc1(cnt, xw)
    h2n = h1p.reshape(2 * N, FH)
    p = _agg1(h2n, ei3).reshape(NPAD, D1)
    h2d = _tc2(cnt, p, h1p, b1.reshape(1, D1), W2)
    q = _agg2(h2d.reshape(2 * N, D2), ei3).reshape(NPAD, 2 * D2)
    out = _tc3(cnt, q, h2d, b2.reshape(1, D2))
    return out
